# Initial kernel scaffold; baseline (speedup 1.0000x reference)
import jax
import jax.numpy as jnp
from jax.experimental import pallas as pl


def kernel(atom_features, bond_features, atom_neighbors_1, atom_neighbors_2, atom_neighbors_3, atom_neighbors_4, bond_neighbors_1, bond_neighbors_2, bond_neighbors_3, bond_neighbors_4, mol_ids, W_self0, b_self0, W_deg0_1, b_deg0_1, W_deg0_2, b_deg0_2, W_deg0_3, b_deg0_3, W_deg0_4, b_deg0_4, W_self1, b_self1, W_deg1_1, b_deg1_1, W_deg1_2, b_deg1_2, W_deg1_3, b_deg1_3, W_deg1_4, b_deg1_4, W_out0, b_out0, W_out1, b_out1, W_out2, b_out2):
    raise NotImplementedError("write your pallas kernel here")


# speedup vs baseline: 1.1054x; 1.1054x over previous
import functools

import jax
import jax.numpy as jnp
from jax import lax
from jax.experimental import pallas as pl
from jax.experimental.pallas import tpu as pltpu
from jax.experimental.pallas import tpu_sc as plsc

N_ATOMS = 100000
N_BONDS = 200000
N_MOLS = 4000
DEGS = (1, 2, 3, 4)
CNTS = (20000, 30000, 30000, 20000)
CNTS_PAD = (20096, 30080, 30080, 20096)
STARTS = (0, 20096, 50176, 80256)
NP = 100352
TILES = NP // 128
TILE_BOUNDS = (157, 392, 627)
REAL_ENDS = (20000, 50096, 80176, 100256)
FP = 512
FPAD = 128
BFPAD = 16
FP_ROWS = 4016
CHUNK = 64
NCHUNKS = tuple(c // CHUNK for c in CNTS_PAD)
IDX_OFFS = (0, 20096, 80256, 170496)
NW = 32
SC_MESH = plsc.VectorSubcoreMesh(core_axis_name="c", subcore_axis_name="s")


def _gather_body(tab, idx_hbm, out_hbm, bufs, sem, feat):
    wid = lax.axis_index("c") * 16 + lax.axis_index("s")
    for bi, d in enumerate(DEGS):
        ibuf, rbuf = bufs[2 * bi], bufs[2 * bi + 1]
        obuf = bufs[8]
        nmax = (NCHUNKS[bi] + NW - 1) // NW

        def chunk_body(k, _, bi=bi, d=d, ibuf=ibuf, rbuf=rbuf, obuf=obuf):
            cid = wid + k * NW

            @pl.when(cid < NCHUNKS[bi])
            def _():
                row0 = STARTS[bi] + cid * CHUNK
                pltpu.async_copy(
                    idx_hbm.at[pl.ds(IDX_OFFS[bi] + cid * CHUNK * d,
                                     CHUNK * d)], ibuf, sem).wait()
                pltpu.async_copy(tab.at[ibuf], rbuf, sem).wait()

                def row_body(r, _):
                    for kk in range(feat // 16):
                        acc = rbuf[r * d, pl.ds(kk * 16, 16)]
                        for j in range(1, d):
                            acc = acc + rbuf[r * d + j, pl.ds(kk * 16, 16)]
                        obuf[r, pl.ds(kk * 16, 16)] = acc
                    return 0

                lax.fori_loop(0, CHUNK, row_body, 0)
                pltpu.async_copy(obuf, out_hbm.at[pl.ds(row0, CHUNK)],
                                 sem).wait()
            return 0

        lax.fori_loop(0, nmax, chunk_body, 0)


def _gather_scratch(feat):
    scratch = []
    for d in DEGS:
        scratch.append(pltpu.VMEM((CHUNK * d,), jnp.int32))
        scratch.append(pltpu.VMEM((CHUNK * d, feat), jnp.float32))
    scratch.append(pltpu.VMEM((CHUNK, feat), jnp.float32))
    scratch.append(pltpu.SemaphoreType.DMA)
    return tuple(scratch)


def _atom_gather(table, aidx):
    def body(tab, idx, out, *rest):
        _gather_body(tab, idx, out, rest[:9], rest[9], FPAD)

    fn = pl.kernel(body, out_type=jax.ShapeDtypeStruct((NP, FPAD),
                                                       jnp.float32),
                   mesh=SC_MESH, scratch_types=_gather_scratch(FPAD))
    return fn(table, aidx)


def _bond_gather(btable, bidx):
    def body(tab, idx, out, *rest):
        _gather_body(tab, idx, out, rest[:9], rest[9], BFPAD)

    fn = pl.kernel(body, out_type=jax.ShapeDtypeStruct((NP, BFPAD),
                                                       jnp.float32),
                   mesh=SC_MESH, scratch_types=_gather_scratch(BFPAD),
                   compiler_params=pltpu.CompilerParams(
                       use_tc_tiling_on_sc=False))
    return fn(btable, bidx)


HFP = FP // 2


def _fp_scatter(o0, o1, o2, ids):
    chunks_per_s = NP // CHUNK // 16

    def body(o0r, o1r, o2r, idr, fp_out, obuf, idbuf, zbuf, fp_sh, sem):
        c = lax.axis_index("c")
        s = lax.axis_index("s")
        col0 = c * HFP

        z = jnp.zeros((16,), jnp.float32)

        def zrow(r, _):
            for kk in range(HFP // 16):
                zbuf[r, pl.ds(kk * 16, 16)] = z
            return 0

        lax.fori_loop(0, 128, zrow, 0)
        base = s * 256

        @pl.when(s < 15)
        def _():
            pltpu.async_copy(zbuf, fp_sh.at[pl.ds(base, 128)], sem).wait()
            pltpu.async_copy(zbuf, fp_sh.at[pl.ds(base + 128, 128)],
                             sem).wait()

        @pl.when(s == 15)
        def _():
            pltpu.async_copy(zbuf, fp_sh.at[pl.ds(3840, 128)], sem).wait()
            pltpu.async_copy(zbuf.at[pl.ds(0, 48)],
                             fp_sh.at[pl.ds(3968, 48)], sem).wait()

        plsc.subcore_barrier()

        def chunk_body(k, _):
            row0 = (s * chunks_per_s + k) * CHUNK
            pltpu.async_copy(idr.at[pl.ds(row0, CHUNK)], idbuf, sem).wait()
            for o_hbm in (o0r, o1r, o2r):
                pltpu.async_copy(
                    o_hbm.at[pl.ds(row0, CHUNK), pl.ds(col0, HFP)], obuf,
                    sem).wait()
                pltpu.sync_copy(obuf, fp_sh.at[idbuf], add=True)
            return 0

        lax.fori_loop(0, chunks_per_s, chunk_body, 0)
        plsc.subcore_barrier()

        @pl.when(s < 15)
        def _():
            pltpu.async_copy(fp_sh.at[pl.ds(base, 256)],
                             fp_out.at[c].at[pl.ds(base, 256)], sem).wait()

        @pl.when(s == 15)
        def _():
            pltpu.async_copy(fp_sh.at[pl.ds(3840, 176)],
                             fp_out.at[c].at[pl.ds(3840, 176)], sem).wait()

    fn = pl.kernel(
        body,
        out_type=jax.ShapeDtypeStruct((2, FP_ROWS, HFP), jnp.float32),
        mesh=SC_MESH,
        scratch_types=(
            pltpu.VMEM((CHUNK, HFP), jnp.float32),
            pltpu.VMEM((CHUNK,), jnp.int32),
            pltpu.VMEM((128, HFP), jnp.float32),
            pltpu.VMEM_SHARED((FP_ROWS, HFP), jnp.float32),
            pltpu.SemaphoreType.DMA,
        ),
        compiler_params=pltpu.CompilerParams(use_tc_tiling_on_sc=False),
    )
    return fn(o0, o1, o2, ids)


def _bucket_of(t):
    return ((t >= TILE_BOUNDS[0]).astype(jnp.int32)
            + (t >= TILE_BOUNDS[1]).astype(jnp.int32)
            + (t >= TILE_BOUNDS[2]).astype(jnp.int32))


def _layer_body(x_ref, gx_ref, gb_ref, woutT, bout, wselfT, bself, wxT, wbT,
                bdeg, o_ref, xn_ref):
    t = pl.program_id(0)
    x = x_ref[...]
    logits = jnp.dot(x, woutT[...], preferred_element_type=jnp.float32)
    logits = logits + bout[...]
    m = jnp.max(logits, axis=1, keepdims=True)
    e = jnp.exp(logits - m)
    o_ref[...] = e / jnp.sum(e, axis=1, keepdims=True)

    selfl = jnp.dot(x, wselfT[...], preferred_element_type=jnp.float32)
    nb = jnp.dot(gx_ref[...], wxT[0], preferred_element_type=jnp.float32)
    nb = nb + jnp.dot(gb_ref[...], wbT[0], preferred_element_type=jnp.float32)
    tot = nb + bdeg[0] + selfl + bself[...]
    nrm = jnp.sqrt(jnp.sum(tot * tot, axis=1, keepdims=True))
    xn = jnp.maximum(tot / jnp.maximum(nrm, 1e-12), 0.0)

    row = t * 128 + lax.broadcasted_iota(jnp.int32, (128, 1), 0)
    b = _bucket_of(t)
    re = jnp.where(b == 0, REAL_ENDS[0],
                   jnp.where(b == 1, REAL_ENDS[1],
                             jnp.where(b == 2, REAL_ENDS[2], REAL_ENDS[3])))
    xn_ref[...] = jnp.where(row < re, xn, 0.0)


def _dense_layer(x, gx, gb, woutT, bout, wselfT, bself, wxT, wbT, bdeg):
    wmap = lambda t: (_bucket_of(t), 0, 0)
    return pl.pallas_call(
        _layer_body,
        grid=(TILES,),
        in_specs=[
            pl.BlockSpec((128, FPAD), lambda t: (t, 0)),
            pl.BlockSpec((128, FPAD), lambda t: (t, 0)),
            pl.BlockSpec((128, BFPAD), lambda t: (t, 0)),
            pl.BlockSpec((FPAD, FP), lambda t: (0, 0)),
            pl.BlockSpec((1, FP), lambda t: (0, 0)),
            pl.BlockSpec((FPAD, FPAD), lambda t: (0, 0)),
            pl.BlockSpec((1, FPAD), lambda t: (0, 0)),
            pl.BlockSpec((1, FPAD, FPAD), wmap),
            pl.BlockSpec((1, BFPAD, FPAD), wmap),
            pl.BlockSpec((1, 1, FPAD), wmap),
        ],
        out_specs=[
            pl.BlockSpec((128, FP), lambda t: (t, 0)),
            pl.BlockSpec((128, FPAD), lambda t: (t, 0)),
        ],
        out_shape=[
            jax.ShapeDtypeStruct((NP, FP), jnp.float32),
            jax.ShapeDtypeStruct((NP, FPAD), jnp.float32),
        ],
    )(x, gx, gb, woutT, bout, wselfT, bself, wxT, wbT, bdeg)


def _out_body(x_ref, woutT, bout, o_ref):
    logits = jnp.dot(x_ref[...], woutT[...], preferred_element_type=jnp.float32)
    logits = logits + bout[...]
    m = jnp.max(logits, axis=1, keepdims=True)
    e = jnp.exp(logits - m)
    o_ref[...] = e / jnp.sum(e, axis=1, keepdims=True)


def _dense_out(x, woutT, bout):
    return pl.pallas_call(
        _out_body,
        grid=(TILES,),
        in_specs=[
            pl.BlockSpec((128, FPAD), lambda t: (t, 0)),
            pl.BlockSpec((FPAD, FP), lambda t: (0, 0)),
            pl.BlockSpec((1, FP), lambda t: (0, 0)),
        ],
        out_specs=pl.BlockSpec((128, FP), lambda t: (t, 0)),
        out_shape=jax.ShapeDtypeStruct((NP, FP), jnp.float32),
    )(x, woutT, bout)


def _remap(a):
    a = a.astype(jnp.int32)
    return (a + 96 * (a >= 20000).astype(jnp.int32)
            + 80 * (a >= 50000).astype(jnp.int32)
            + 80 * (a >= 80000).astype(jnp.int32))


def _pad_rows(parts, fills):
    segs = []
    for part, fill in zip(parts, fills):
        segs.append(part)
        segs.append(fill)
    return jnp.concatenate(segs, axis=0)


def _pad_atom_rows(a, fill_val=0.0):
    f = a.shape[1]
    parts = [a[0:20000], a[20000:50000], a[50000:80000], a[80000:100000]]
    fills = [jnp.full((96, f), fill_val, a.dtype),
             jnp.full((80, f), fill_val, a.dtype),
             jnp.full((80, f), fill_val, a.dtype),
             jnp.full((96, f), fill_val, a.dtype)]
    return _pad_rows(parts, fills)


def _flat_idx(idx_list, remap):
    segs = []
    for d, cnt, cnt_pad in zip(DEGS, CNTS, CNTS_PAD):
        idx = idx_list[d - 1].astype(jnp.int32)
        if remap:
            idx = _remap(idx)
        idx = jnp.concatenate(
            [idx, jnp.zeros((cnt_pad - cnt, d), jnp.int32)], axis=0)
        segs.append(idx.reshape(-1))
    return jnp.concatenate(segs, axis=0)


def _prep_wout(w, b, f):
    wt = jnp.zeros((FPAD, FP), jnp.float32).at[:f].set(w.T)
    return wt, b.reshape(1, FP)


def _prep_layer(pd, i, f):
    woutT, bout = _prep_wout(pd["W_out%d" % i], pd["b_out%d" % i], f)
    wselfT = jnp.zeros((FPAD, FPAD), jnp.float32).at[:f, :100].set(
        pd["W_self%d" % i].T)
    bself = jnp.zeros((1, FPAD), jnp.float32).at[0, :100].set(
        pd["b_self%d" % i])
    wx, wb, bd = [], [], []
    for d in DEGS:
        w = pd["W_deg%d_%d" % (i, d)]
        wx.append(jnp.zeros((FPAD, FPAD), jnp.float32).at[:f, :100].set(
            w[:, :f].T))
        wb.append(jnp.zeros((BFPAD, FPAD), jnp.float32).at[:6, :100].set(
            w[:, f:].T))
        bd.append(jnp.zeros((1, FPAD), jnp.float32).at[0, :100].set(
            pd["b_deg%d_%d" % (i, d)]))
    return (woutT, bout, wselfT, bself, jnp.stack(wx), jnp.stack(wb),
            jnp.stack(bd))


def kernel(atom_features, bond_features, atom_neighbors_1, atom_neighbors_2,
           atom_neighbors_3, atom_neighbors_4, bond_neighbors_1,
           bond_neighbors_2, bond_neighbors_3, bond_neighbors_4, mol_ids,
           W_self0, b_self0, W_deg0_1, b_deg0_1, W_deg0_2, b_deg0_2,
           W_deg0_3, b_deg0_3, W_deg0_4, b_deg0_4, W_self1, b_self1,
           W_deg1_1, b_deg1_1, W_deg1_2, b_deg1_2, W_deg1_3, b_deg1_3,
           W_deg1_4, b_deg1_4, W_out0, b_out0, W_out1, b_out1, W_out2,
           b_out2):
    pd = dict(W_self0=W_self0, b_self0=b_self0, W_self1=W_self1,
              b_self1=b_self1, W_out0=W_out0, b_out0=b_out0, W_out1=W_out1,
              b_out1=b_out1, W_out2=W_out2, b_out2=b_out2,
              W_deg0_1=W_deg0_1, b_deg0_1=b_deg0_1, W_deg0_2=W_deg0_2,
              b_deg0_2=b_deg0_2, W_deg0_3=W_deg0_3, b_deg0_3=b_deg0_3,
              W_deg0_4=W_deg0_4, b_deg0_4=b_deg0_4,
              W_deg1_1=W_deg1_1, b_deg1_1=b_deg1_1, W_deg1_2=W_deg1_2,
              b_deg1_2=b_deg1_2, W_deg1_3=W_deg1_3, b_deg1_3=b_deg1_3,
              W_deg1_4=W_deg1_4, b_deg1_4=b_deg1_4)

    x0 = jnp.pad(_pad_atom_rows(atom_features), ((0, 0), (0, FPAD - 62)))
    bf = jnp.pad(bond_features, ((0, 0), (0, BFPAD - 6)))
    aidx = _flat_idx([atom_neighbors_1, atom_neighbors_2, atom_neighbors_3,
                      atom_neighbors_4], remap=True)
    bidx = _flat_idx([bond_neighbors_1, bond_neighbors_2, bond_neighbors_3,
                      bond_neighbors_4], remap=False)
    pad_ids = [4000 + (jnp.arange(n, dtype=jnp.int32) % 16)
               for n in (96, 80, 80, 96)]
    mi = mol_ids.astype(jnp.int32)
    ids = _pad_rows([mi[0:20000], mi[20000:50000], mi[50000:80000],
                     mi[80000:100000]], pad_ids)

    p0 = _prep_layer(pd, 0, 62)
    p1 = _prep_layer(pd, 1, 100)
    woutT2, bout2 = _prep_wout(W_out2, b_out2, 100)

    gb = _bond_gather(bf, bidx)
    gx0 = _atom_gather(x0, aidx)
    o0, x1 = _dense_layer(x0, gx0, gb, *p0)
    gx1 = _atom_gather(x1, aidx)
    o1, x2 = _dense_layer(x1, gx1, gb, *p1)
    o2 = _dense_out(x2, woutT2, bout2)

    fp2 = _fp_scatter(o0, o1, o2, ids)
    return jnp.concatenate([fp2[0, :N_MOLS], fp2[1, :N_MOLS]], axis=1)


# speedup vs baseline: 1.2500x; 1.1308x over previous
import functools

import jax
import jax.numpy as jnp
from jax import lax
from jax.experimental import pallas as pl
from jax.experimental.pallas import tpu as pltpu
from jax.experimental.pallas import tpu_sc as plsc

N_ATOMS = 100000
N_BONDS = 200000
N_MOLS = 4000
DEGS = (1, 2, 3, 4)
CNTS = (20000, 30000, 30000, 20000)
CNTS_PAD = (20096, 30080, 30080, 20096)
STARTS = (0, 20096, 50176, 80256)
NP = 100352
TILES = NP // 128
TILE_BOUNDS = (157, 392, 627)
REAL_ENDS = (20000, 50096, 80176, 100256)
FP = 512
FPAD = 128
BFPAD = 16
FP_ROWS = 4016
CHUNK = 64
NCHUNKS = tuple(c // CHUNK for c in CNTS_PAD)
IDX_OFFS = (0, 20096, 80256, 170496)
NW = 32
SC_MESH = plsc.VectorSubcoreMesh(core_axis_name="c", subcore_axis_name="s")


def _gather_body(tab, idx_hbm, out_hbm, bufs, sem, feat):
    wid = lax.axis_index("c") * 16 + lax.axis_index("s")
    for bi, d in enumerate(DEGS):
        ibuf, rbuf = bufs[2 * bi], bufs[2 * bi + 1]
        obuf = bufs[8]
        nmax = (NCHUNKS[bi] + NW - 1) // NW

        def chunk_body(k, _, bi=bi, d=d, ibuf=ibuf, rbuf=rbuf, obuf=obuf):
            cid = wid + k * NW

            @pl.when(cid < NCHUNKS[bi])
            def _():
                row0 = STARTS[bi] + cid * CHUNK
                pltpu.async_copy(
                    idx_hbm.at[pl.ds(IDX_OFFS[bi] + cid * CHUNK * d,
                                     CHUNK * d)], ibuf, sem).wait()
                pltpu.async_copy(tab.at[ibuf], rbuf, sem).wait()

                def row_body(r, _):
                    for kk in range(feat // 16):
                        acc = rbuf[r * d, pl.ds(kk * 16, 16)]
                        for j in range(1, d):
                            acc = acc + rbuf[r * d + j, pl.ds(kk * 16, 16)]
                        obuf[r, pl.ds(kk * 16, 16)] = acc
                    return 0

                lax.fori_loop(0, CHUNK, row_body, 0)
                pltpu.async_copy(obuf, out_hbm.at[pl.ds(row0, CHUNK)],
                                 sem).wait()
            return 0

        lax.fori_loop(0, nmax, chunk_body, 0)


def _gather_scratch(feat):
    scratch = []
    for d in DEGS:
        scratch.append(pltpu.VMEM((CHUNK * d,), jnp.int32))
        scratch.append(pltpu.VMEM((CHUNK * d, feat), jnp.float32))
    scratch.append(pltpu.VMEM((CHUNK, feat), jnp.float32))
    scratch.append(pltpu.SemaphoreType.DMA)
    return tuple(scratch)


def _atom_gather(table, aidx):
    def body(tab, idx, out, *rest):
        _gather_body(tab, idx, out, rest[:9], rest[9], FPAD)

    fn = pl.kernel(body, out_type=jax.ShapeDtypeStruct((NP, FPAD),
                                                       jnp.float32),
                   mesh=SC_MESH, scratch_types=_gather_scratch(FPAD))
    return fn(table, aidx)


def _bond_gather(btable, bidx):
    def body(tab, idx, out, *rest):
        _gather_body(tab, idx, out, rest[:9], rest[9], BFPAD)

    fn = pl.kernel(body, out_type=jax.ShapeDtypeStruct((NP, BFPAD),
                                                       jnp.float32),
                   mesh=SC_MESH, scratch_types=_gather_scratch(BFPAD),
                   compiler_params=pltpu.CompilerParams(
                       use_tc_tiling_on_sc=False))
    return fn(btable, bidx)


SCHUNK = 128


def _fp_scatter(o0, o1, o2, ids):
    chunks_per_s = NP // SCHUNK // 16

    def body(o0r, o1r, o2r, idr, fp_out, ob0, ob1, idbuf, zbuf, fa, fb, sem):
        c = lax.axis_index("c")
        s = lax.axis_index("s")

        z = jnp.zeros((16,), jnp.float32)

        def zrow(r, _):
            for kk in range(128 // 16):
                zbuf[r, pl.ds(kk * 16, 16)] = z
            return 0

        lax.fori_loop(0, 128, zrow, 0)
        base = s * 256
        for fp_sh in (fa, fb):
            @pl.when(s < 15)
            def _(fp_sh=fp_sh):
                pltpu.async_copy(zbuf, fp_sh.at[pl.ds(base, 128)], sem).wait()
                pltpu.async_copy(zbuf, fp_sh.at[pl.ds(base + 128, 128)],
                                 sem).wait()

            @pl.when(s == 15)
            def _(fp_sh=fp_sh):
                pltpu.async_copy(zbuf, fp_sh.at[pl.ds(3840, 128)], sem).wait()
                pltpu.async_copy(zbuf.at[pl.ds(0, 48)],
                                 fp_sh.at[pl.ds(3968, 48)], sem).wait()

        plsc.subcore_barrier()

        srcs = []
        for o_hbm in (o0r, o1r, o2r):
            for local, fp_sh in ((0, fa), (1, fb)):
                srcs.append((o_hbm, local, fp_sh))

        def load(o_hbm, local, row0, buf):
            return pltpu.async_copy(o_hbm.at[2 * c + local].at[
                pl.ds(row0, SCHUNK)], buf, sem)

        def chunk_body(k, _):
            row0 = (s * chunks_per_s + k) * SCHUNK
            pltpu.async_copy(idr.at[pl.ds(row0, SCHUNK)], idbuf, sem).wait()
            bufs = (ob0, ob1)
            load(srcs[0][0], srcs[0][1], row0, bufs[0]).wait()
            for i, (o_hbm, local, fp_sh) in enumerate(srcs):
                nxt = None
                if i + 1 < len(srcs):
                    nxt = load(srcs[i + 1][0], srcs[i + 1][1], row0,
                               bufs[(i + 1) % 2])
                pltpu.sync_copy(bufs[i % 2], fp_sh.at[idbuf], add=True)
                if nxt is not None:
                    nxt.wait()
            return 0

        lax.fori_loop(0, chunks_per_s, chunk_body, 0)
        plsc.subcore_barrier()

        for local, fp_sh in ((0, fa), (1, fb)):
            @pl.when(s < 15)
            def _(local=local, fp_sh=fp_sh):
                pltpu.async_copy(fp_sh.at[pl.ds(base, 256)],
                                 fp_out.at[2 * c + local].at[
                                     pl.ds(base, 256)], sem).wait()

            @pl.when(s == 15)
            def _(local=local, fp_sh=fp_sh):
                pltpu.async_copy(fp_sh.at[pl.ds(3840, 176)],
                                 fp_out.at[2 * c + local].at[
                                     pl.ds(3840, 176)], sem).wait()

    fn = pl.kernel(
        body,
        out_type=jax.ShapeDtypeStruct((4, FP_ROWS, 128), jnp.float32),
        mesh=SC_MESH,
        scratch_types=(
            pltpu.VMEM((SCHUNK, 128), jnp.float32),
            pltpu.VMEM((SCHUNK, 128), jnp.float32),
            pltpu.VMEM((SCHUNK,), jnp.int32),
            pltpu.VMEM((128, 128), jnp.float32),
            pltpu.VMEM_SHARED((FP_ROWS, 128), jnp.float32),
            pltpu.VMEM_SHARED((FP_ROWS, 128), jnp.float32),
            pltpu.SemaphoreType.DMA,
        ),
        compiler_params=pltpu.CompilerParams(use_tc_tiling_on_sc=False),
    )
    return fn(o0, o1, o2, ids)


def _bucket_of(t):
    return ((t >= TILE_BOUNDS[0]).astype(jnp.int32)
            + (t >= TILE_BOUNDS[1]).astype(jnp.int32)
            + (t >= TILE_BOUNDS[2]).astype(jnp.int32))


def _layer_body(x_ref, gx_ref, gb_ref, woutT, bout, wselfT, bself, wxT, wbT,
                bdeg, o_ref, xn_ref):
    t = pl.program_id(0)
    x = x_ref[...]
    logits = jnp.dot(x, woutT[...], preferred_element_type=jnp.float32)
    logits = logits + bout[...]
    m = jnp.max(logits, axis=1, keepdims=True)
    e = jnp.exp(logits - m)
    o = e / jnp.sum(e, axis=1, keepdims=True)
    for k in range(4):
        o_ref[k] = o[:, k * 128:(k + 1) * 128]

    selfl = jnp.dot(x, wselfT[...], preferred_element_type=jnp.float32)
    nb = jnp.dot(gx_ref[...], wxT[0], preferred_element_type=jnp.float32)
    nb = nb + jnp.dot(gb_ref[...], wbT[0], preferred_element_type=jnp.float32)
    tot = nb + bdeg[0] + selfl + bself[...]
    nrm = jnp.sqrt(jnp.sum(tot * tot, axis=1, keepdims=True))
    xn = jnp.maximum(tot / jnp.maximum(nrm, 1e-12), 0.0)

    row = t * 128 + lax.broadcasted_iota(jnp.int32, (128, 1), 0)
    b = _bucket_of(t)
    re = jnp.where(b == 0, REAL_ENDS[0],
                   jnp.where(b == 1, REAL_ENDS[1],
                             jnp.where(b == 2, REAL_ENDS[2], REAL_ENDS[3])))
    xn_ref[...] = jnp.where(row < re, xn, 0.0)


def _dense_layer(x, gx, gb, woutT, bout, wselfT, bself, wxT, wbT, bdeg):
    wmap = lambda t: (_bucket_of(t), 0, 0)
    return pl.pallas_call(
        _layer_body,
        grid=(TILES,),
        in_specs=[
            pl.BlockSpec((128, FPAD), lambda t: (t, 0)),
            pl.BlockSpec((128, FPAD), lambda t: (t, 0)),
            pl.BlockSpec((128, BFPAD), lambda t: (t, 0)),
            pl.BlockSpec((FPAD, FP), lambda t: (0, 0)),
            pl.BlockSpec((1, FP), lambda t: (0, 0)),
            pl.BlockSpec((FPAD, FPAD), lambda t: (0, 0)),
            pl.BlockSpec((1, FPAD), lambda t: (0, 0)),
            pl.BlockSpec((1, FPAD, FPAD), wmap),
            pl.BlockSpec((1, BFPAD, FPAD), wmap),
            pl.BlockSpec((1, 1, FPAD), wmap),
        ],
        out_specs=[
            pl.BlockSpec((4, 128, 128), lambda t: (0, t, 0)),
            pl.BlockSpec((128, FPAD), lambda t: (t, 0)),
        ],
        out_shape=[
            jax.ShapeDtypeStruct((4, NP, 128), jnp.float32),
            jax.ShapeDtypeStruct((NP, FPAD), jnp.float32),
        ],
    )(x, gx, gb, woutT, bout, wselfT, bself, wxT, wbT, bdeg)


def _out_body(x_ref, woutT, bout, o_ref):
    logits = jnp.dot(x_ref[...], woutT[...], preferred_element_type=jnp.float32)
    logits = logits + bout[...]
    m = jnp.max(logits, axis=1, keepdims=True)
    e = jnp.exp(logits - m)
    o = e / jnp.sum(e, axis=1, keepdims=True)
    for k in range(4):
        o_ref[k] = o[:, k * 128:(k + 1) * 128]


def _dense_out(x, woutT, bout):
    return pl.pallas_call(
        _out_body,
        grid=(TILES,),
        in_specs=[
            pl.BlockSpec((128, FPAD), lambda t: (t, 0)),
            pl.BlockSpec((FPAD, FP), lambda t: (0, 0)),
            pl.BlockSpec((1, FP), lambda t: (0, 0)),
        ],
        out_specs=pl.BlockSpec((4, 128, 128), lambda t: (0, t, 0)),
        out_shape=jax.ShapeDtypeStruct((4, NP, 128), jnp.float32),
    )(x, woutT, bout)


def _remap(a):
    a = a.astype(jnp.int32)
    return (a + 96 * (a >= 20000).astype(jnp.int32)
            + 80 * (a >= 50000).astype(jnp.int32)
            + 80 * (a >= 80000).astype(jnp.int32))


def _pad_rows(parts, fills):
    segs = []
    for part, fill in zip(parts, fills):
        segs.append(part)
        segs.append(fill)
    return jnp.concatenate(segs, axis=0)


def _pad_atom_rows(a, fill_val=0.0):
    f = a.shape[1]
    parts = [a[0:20000], a[20000:50000], a[50000:80000], a[80000:100000]]
    fills = [jnp.full((96, f), fill_val, a.dtype),
             jnp.full((80, f), fill_val, a.dtype),
             jnp.full((80, f), fill_val, a.dtype),
             jnp.full((96, f), fill_val, a.dtype)]
    return _pad_rows(parts, fills)


def _flat_idx(idx_list, remap):
    segs = []
    for d, cnt, cnt_pad in zip(DEGS, CNTS, CNTS_PAD):
        idx = idx_list[d - 1].astype(jnp.int32)
        if remap:
            idx = _remap(idx)
        idx = jnp.concatenate(
            [idx, jnp.zeros((cnt_pad - cnt, d), jnp.int32)], axis=0)
        segs.append(idx.reshape(-1))
    return jnp.concatenate(segs, axis=0)


def _prep_wout(w, b, f):
    wt = jnp.zeros((FPAD, FP), jnp.float32).at[:f].set(w.T)
    return wt, b.reshape(1, FP)


def _prep_layer(pd, i, f):
    woutT, bout = _prep_wout(pd["W_out%d" % i], pd["b_out%d" % i], f)
    wselfT = jnp.zeros((FPAD, FPAD), jnp.float32).at[:f, :100].set(
        pd["W_self%d" % i].T)
    bself = jnp.zeros((1, FPAD), jnp.float32).at[0, :100].set(
        pd["b_self%d" % i])
    wx, wb, bd = [], [], []
    for d in DEGS:
        w = pd["W_deg%d_%d" % (i, d)]
        wx.append(jnp.zeros((FPAD, FPAD), jnp.float32).at[:f, :100].set(
            w[:, :f].T))
        wb.append(jnp.zeros((BFPAD, FPAD), jnp.float32).at[:6, :100].set(
            w[:, f:].T))
        bd.append(jnp.zeros((1, FPAD), jnp.float32).at[0, :100].set(
            pd["b_deg%d_%d" % (i, d)]))
    return (woutT, bout, wselfT, bself, jnp.stack(wx), jnp.stack(wb),
            jnp.stack(bd))


def kernel(atom_features, bond_features, atom_neighbors_1, atom_neighbors_2,
           atom_neighbors_3, atom_neighbors_4, bond_neighbors_1,
           bond_neighbors_2, bond_neighbors_3, bond_neighbors_4, mol_ids,
           W_self0, b_self0, W_deg0_1, b_deg0_1, W_deg0_2, b_deg0_2,
           W_deg0_3, b_deg0_3, W_deg0_4, b_deg0_4, W_self1, b_self1,
           W_deg1_1, b_deg1_1, W_deg1_2, b_deg1_2, W_deg1_3, b_deg1_3,
           W_deg1_4, b_deg1_4, W_out0, b_out0, W_out1, b_out1, W_out2,
           b_out2):
    pd = dict(W_self0=W_self0, b_self0=b_self0, W_self1=W_self1,
              b_self1=b_self1, W_out0=W_out0, b_out0=b_out0, W_out1=W_out1,
              b_out1=b_out1, W_out2=W_out2, b_out2=b_out2,
              W_deg0_1=W_deg0_1, b_deg0_1=b_deg0_1, W_deg0_2=W_deg0_2,
              b_deg0_2=b_deg0_2, W_deg0_3=W_deg0_3, b_deg0_3=b_deg0_3,
              W_deg0_4=W_deg0_4, b_deg0_4=b_deg0_4,
              W_deg1_1=W_deg1_1, b_deg1_1=b_deg1_1, W_deg1_2=W_deg1_2,
              b_deg1_2=b_deg1_2, W_deg1_3=W_deg1_3, b_deg1_3=b_deg1_3,
              W_deg1_4=W_deg1_4, b_deg1_4=b_deg1_4)

    x0 = jnp.pad(_pad_atom_rows(atom_features), ((0, 0), (0, FPAD - 62)))
    bf = jnp.pad(bond_features, ((0, 0), (0, BFPAD - 6)))
    aidx = _flat_idx([atom_neighbors_1, atom_neighbors_2, atom_neighbors_3,
                      atom_neighbors_4], remap=True)
    bidx = _flat_idx([bond_neighbors_1, bond_neighbors_2, bond_neighbors_3,
                      bond_neighbors_4], remap=False)
    pad_ids = [4000 + (jnp.arange(n, dtype=jnp.int32) % 16)
               for n in (96, 80, 80, 96)]
    mi = mol_ids.astype(jnp.int32)
    ids = _pad_rows([mi[0:20000], mi[20000:50000], mi[50000:80000],
                     mi[80000:100000]], pad_ids)

    p0 = _prep_layer(pd, 0, 62)
    p1 = _prep_layer(pd, 1, 100)
    woutT2, bout2 = _prep_wout(W_out2, b_out2, 100)

    gb = _bond_gather(bf, bidx)
    gx0 = _atom_gather(x0, aidx)
    o0, x1 = _dense_layer(x0, gx0, gb, *p0)
    gx1 = _atom_gather(x1, aidx)
    o1, x2 = _dense_layer(x1, gx1, gb, *p1)
    o2 = _dense_out(x2, woutT2, bout2)

    fp4 = _fp_scatter(o0, o1, o2, ids)
    return jnp.concatenate([fp4[k, :N_MOLS] for k in range(4)], axis=1)


# speedup vs baseline: 1.3091x; 1.0473x over previous
import functools

import jax
import jax.numpy as jnp
from jax import lax
from jax.experimental import pallas as pl
from jax.experimental.pallas import tpu as pltpu
from jax.experimental.pallas import tpu_sc as plsc

N_ATOMS = 100000
N_BONDS = 200000
N_MOLS = 4000
DEGS = (1, 2, 3, 4)
CNTS = (20000, 30000, 30000, 20000)
CNTS_PAD = (20096, 30080, 30080, 20096)
STARTS = (0, 20096, 50176, 80256)
NP = 100352
TILES = NP // 128
TILE_BOUNDS = (157, 392, 627)
REAL_ENDS = (20000, 50096, 80176, 100256)
FP = 512
FPAD = 128
BFPAD = 16
FP_ROWS = 4016
CHUNK = 64
NCHUNKS = tuple(c // CHUNK for c in CNTS_PAD)
IDX_OFFS = (0, 20096, 80256, 170496)
NW = 32
SC_MESH = plsc.VectorSubcoreMesh(core_axis_name="c", subcore_axis_name="s")


def _gather_body(tab, idx_hbm, out_hbm, ibufs, rbufs, obufs, semi, semg,
                 semo, feat):
    wid = lax.axis_index("c") * 16 + lax.axis_index("s")
    for bi, d in enumerate(DEGS):
        nmax = (NCHUNKS[bi] + NW - 1) // NW
        nrows = CHUNK * d

        def valid(k):
            return wid + k * NW < NCHUNKS[bi]

        def cid(k):
            return wid + k * NW

        def idx_copy(k, p):
            return pltpu.make_async_copy(
                idx_hbm.at[pl.ds(IDX_OFFS[bi] + cid(k) * nrows, nrows)],
                ibufs[2 * bi + p], semi)

        def gat_copy(k, p):
            return pltpu.make_async_copy(tab.at[ibufs[2 * bi + p]],
                                         rbufs[p].at[pl.ds(0, nrows)], semg)

        def out_copy(k, p):
            return pltpu.make_async_copy(
                obufs[p], out_hbm.at[pl.ds(STARTS[bi] + cid(k) * CHUNK,
                                           CHUNK)], semo)

        @pl.when(valid(0))
        def _():
            idx_copy(0, 0).start()
            idx_copy(0, 0).wait()
            gat_copy(0, 0).start()

        for k in range(nmax):
            p = k % 2
            q = (k + 1) % 2
            if k + 1 < nmax:
                @pl.when(valid(k + 1))
                def _(k=k, q=q):
                    idx_copy(k + 1, q).start()

            @pl.when(valid(k))
            def _(k=k, p=p):
                gat_copy(k, p).wait()

            if k + 1 < nmax:
                @pl.when(valid(k + 1))
                def _(k=k, q=q):
                    idx_copy(k + 1, q).wait()
                    gat_copy(k + 1, q).start()

            @pl.when(valid(k))
            def _(k=k, p=p, d=d):
                if k >= 2:
                    out_copy(k - 2, p).wait()
                rbuf = rbufs[p]
                obuf = obufs[p]

                def row_body(r, _):
                    for kk in range(feat // 16):
                        acc = rbuf[r * d, pl.ds(kk * 16, 16)]
                        for j in range(1, d):
                            acc = acc + rbuf[r * d + j, pl.ds(kk * 16, 16)]
                        obuf[r, pl.ds(kk * 16, 16)] = acc
                    return 0

                lax.fori_loop(0, CHUNK, row_body, 0)
                out_copy(k, p).start()

        for k in (nmax - 2, nmax - 1):
            if k >= 0:
                @pl.when(valid(k))
                def _(k=k):
                    out_copy(k, k % 2).wait()


def _gather_scratch(feat):
    scratch = []
    for d in DEGS:
        for _ in range(2):
            scratch.append(pltpu.VMEM((CHUNK * d,), jnp.int32))
    for _ in range(2):
        scratch.append(pltpu.VMEM((CHUNK * DEGS[-1], feat), jnp.float32))
    for _ in range(2):
        scratch.append(pltpu.VMEM((CHUNK, feat), jnp.float32))
    scratch.append(pltpu.SemaphoreType.DMA)
    scratch.append(pltpu.SemaphoreType.DMA)
    scratch.append(pltpu.SemaphoreType.DMA)
    return tuple(scratch)


def _split_gather_refs(rest):
    ibufs = rest[:8]
    rbufs = rest[8:10]
    obufs = rest[10:12]
    semi, semg, semo = rest[12:15]
    return ibufs, rbufs, obufs, semi, semg, semo


def _atom_gather(table, aidx):
    def body(tab, idx, out, *rest):
        _gather_body(tab, idx, out, *_split_gather_refs(rest), FPAD)

    fn = pl.kernel(body, out_type=jax.ShapeDtypeStruct((NP, FPAD),
                                                       jnp.float32),
                   mesh=SC_MESH, scratch_types=_gather_scratch(FPAD))
    return fn(table, aidx)


def _bond_gather(btable, bidx):
    def body(tab, idx, out, *rest):
        _gather_body(tab, idx, out, *_split_gather_refs(rest), BFPAD)

    fn = pl.kernel(body, out_type=jax.ShapeDtypeStruct((NP, BFPAD),
                                                       jnp.float32),
                   mesh=SC_MESH, scratch_types=_gather_scratch(BFPAD),
                   compiler_params=pltpu.CompilerParams(
                       use_tc_tiling_on_sc=False))
    return fn(btable, bidx)


SCHUNK = 128


def _fp_scatter(o0, o1, o2, ids):
    chunks_per_s = NP // SCHUNK // 16

    def body(o0r, o1r, o2r, idr, fp_out, ob0, ob1, idbuf, zbuf, fa, fb, sem):
        c = lax.axis_index("c")
        s = lax.axis_index("s")

        z = jnp.zeros((16,), jnp.float32)

        def zrow(r, _):
            for kk in range(128 // 16):
                zbuf[r, pl.ds(kk * 16, 16)] = z
            return 0

        lax.fori_loop(0, 128, zrow, 0)
        base = s * 256
        for fp_sh in (fa, fb):
            @pl.when(s < 15)
            def _(fp_sh=fp_sh):
                pltpu.async_copy(zbuf, fp_sh.at[pl.ds(base, 128)], sem).wait()
                pltpu.async_copy(zbuf, fp_sh.at[pl.ds(base + 128, 128)],
                                 sem).wait()

            @pl.when(s == 15)
            def _(fp_sh=fp_sh):
                pltpu.async_copy(zbuf, fp_sh.at[pl.ds(3840, 128)], sem).wait()
                pltpu.async_copy(zbuf.at[pl.ds(0, 48)],
                                 fp_sh.at[pl.ds(3968, 48)], sem).wait()

        plsc.subcore_barrier()

        srcs = []
        for o_hbm in (o0r, o1r, o2r):
            for local, fp_sh in ((0, fa), (1, fb)):
                srcs.append((o_hbm, local, fp_sh))

        def load(o_hbm, local, row0, buf):
            return pltpu.async_copy(o_hbm.at[2 * c + local].at[
                pl.ds(row0, SCHUNK)], buf, sem)

        def chunk_body(k, _):
            row0 = (s * chunks_per_s + k) * SCHUNK
            pltpu.async_copy(idr.at[pl.ds(row0, SCHUNK)], idbuf, sem).wait()
            bufs = (ob0, ob1)
            load(srcs[0][0], srcs[0][1], row0, bufs[0]).wait()
            for i, (o_hbm, local, fp_sh) in enumerate(srcs):
                nxt = None
                if i + 1 < len(srcs):
                    nxt = load(srcs[i + 1][0], srcs[i + 1][1], row0,
                               bufs[(i + 1) % 2])
                pltpu.sync_copy(bufs[i % 2], fp_sh.at[idbuf], add=True)
                if nxt is not None:
                    nxt.wait()
            return 0

        lax.fori_loop(0, chunks_per_s, chunk_body, 0)
        plsc.subcore_barrier()

        for local, fp_sh in ((0, fa), (1, fb)):
            @pl.when(s < 15)
            def _(local=local, fp_sh=fp_sh):
                pltpu.async_copy(fp_sh.at[pl.ds(base, 256)],
                                 fp_out.at[2 * c + local].at[
                                     pl.ds(base, 256)], sem).wait()

            @pl.when(s == 15)
            def _(local=local, fp_sh=fp_sh):
                pltpu.async_copy(fp_sh.at[pl.ds(3840, 176)],
                                 fp_out.at[2 * c + local].at[
                                     pl.ds(3840, 176)], sem).wait()

    fn = pl.kernel(
        body,
        out_type=jax.ShapeDtypeStruct((4, FP_ROWS, 128), jnp.float32),
        mesh=SC_MESH,
        scratch_types=(
            pltpu.VMEM((SCHUNK, 128), jnp.float32),
            pltpu.VMEM((SCHUNK, 128), jnp.float32),
            pltpu.VMEM((SCHUNK,), jnp.int32),
            pltpu.VMEM((128, 128), jnp.float32),
            pltpu.VMEM_SHARED((FP_ROWS, 128), jnp.float32),
            pltpu.VMEM_SHARED((FP_ROWS, 128), jnp.float32),
            pltpu.SemaphoreType.DMA,
        ),
        compiler_params=pltpu.CompilerParams(use_tc_tiling_on_sc=False),
    )
    return fn(o0, o1, o2, ids)


def _bucket_of(t):
    return ((t >= TILE_BOUNDS[0]).astype(jnp.int32)
            + (t >= TILE_BOUNDS[1]).astype(jnp.int32)
            + (t >= TILE_BOUNDS[2]).astype(jnp.int32))


def _layer_body(x_ref, gx_ref, gb_ref, woutT, bout, wselfT, bself, wxT, wbT,
                bdeg, o_ref, xn_ref):
    t = pl.program_id(0)
    x = x_ref[...]
    logits = jnp.dot(x, woutT[...], preferred_element_type=jnp.float32)
    logits = logits + bout[...]
    m = jnp.max(logits, axis=1, keepdims=True)
    e = jnp.exp(logits - m)
    o = e / jnp.sum(e, axis=1, keepdims=True)
    for k in range(4):
        o_ref[k] = o[:, k * 128:(k + 1) * 128]

    selfl = jnp.dot(x, wselfT[...], preferred_element_type=jnp.float32)
    nb = jnp.dot(gx_ref[...], wxT[0], preferred_element_type=jnp.float32)
    nb = nb + jnp.dot(gb_ref[...], wbT[0], preferred_element_type=jnp.float32)
    tot = nb + bdeg[0] + selfl + bself[...]
    nrm = jnp.sqrt(jnp.sum(tot * tot, axis=1, keepdims=True))
    xn = jnp.maximum(tot / jnp.maximum(nrm, 1e-12), 0.0)

    row = t * 128 + lax.broadcasted_iota(jnp.int32, (128, 1), 0)
    b = _bucket_of(t)
    re = jnp.where(b == 0, REAL_ENDS[0],
                   jnp.where(b == 1, REAL_ENDS[1],
                             jnp.where(b == 2, REAL_ENDS[2], REAL_ENDS[3])))
    xn_ref[...] = jnp.where(row < re, xn, 0.0)


def _dense_layer(x, gx, gb, woutT, bout, wselfT, bself, wxT, wbT, bdeg):
    wmap = lambda t: (_bucket_of(t), 0, 0)
    return pl.pallas_call(
        _layer_body,
        grid=(TILES,),
        in_specs=[
            pl.BlockSpec((128, FPAD), lambda t: (t, 0)),
            pl.BlockSpec((128, FPAD), lambda t: (t, 0)),
            pl.BlockSpec((128, BFPAD), lambda t: (t, 0)),
            pl.BlockSpec((FPAD, FP), lambda t: (0, 0)),
            pl.BlockSpec((1, FP), lambda t: (0, 0)),
            pl.BlockSpec((FPAD, FPAD), lambda t: (0, 0)),
            pl.BlockSpec((1, FPAD), lambda t: (0, 0)),
            pl.BlockSpec((1, FPAD, FPAD), wmap),
            pl.BlockSpec((1, BFPAD, FPAD), wmap),
            pl.BlockSpec((1, 1, FPAD), wmap),
        ],
        out_specs=[
            pl.BlockSpec((4, 128, 128), lambda t: (0, t, 0)),
            pl.BlockSpec((128, FPAD), lambda t: (t, 0)),
        ],
        out_shape=[
            jax.ShapeDtypeStruct((4, NP, 128), jnp.float32),
            jax.ShapeDtypeStruct((NP, FPAD), jnp.float32),
        ],
    )(x, gx, gb, woutT, bout, wselfT, bself, wxT, wbT, bdeg)


def _out_body(x_ref, woutT, bout, o_ref):
    logits = jnp.dot(x_ref[...], woutT[...], preferred_element_type=jnp.float32)
    logits = logits + bout[...]
    m = jnp.max(logits, axis=1, keepdims=True)
    e = jnp.exp(logits - m)
    o = e / jnp.sum(e, axis=1, keepdims=True)
    for k in range(4):
        o_ref[k] = o[:, k * 128:(k + 1) * 128]


def _dense_out(x, woutT, bout):
    return pl.pallas_call(
        _out_body,
        grid=(TILES,),
        in_specs=[
            pl.BlockSpec((128, FPAD), lambda t: (t, 0)),
            pl.BlockSpec((FPAD, FP), lambda t: (0, 0)),
            pl.BlockSpec((1, FP), lambda t: (0, 0)),
        ],
        out_specs=pl.BlockSpec((4, 128, 128), lambda t: (0, t, 0)),
        out_shape=jax.ShapeDtypeStruct((4, NP, 128), jnp.float32),
    )(x, woutT, bout)


def _remap(a):
    a = a.astype(jnp.int32)
    return (a + 96 * (a >= 20000).astype(jnp.int32)
            + 80 * (a >= 50000).astype(jnp.int32)
            + 80 * (a >= 80000).astype(jnp.int32))


def _pad_rows(parts, fills):
    segs = []
    for part, fill in zip(parts, fills):
        segs.append(part)
        segs.append(fill)
    return jnp.concatenate(segs, axis=0)


def _pad_atom_rows(a, fill_val=0.0):
    f = a.shape[1]
    parts = [a[0:20000], a[20000:50000], a[50000:80000], a[80000:100000]]
    fills = [jnp.full((96, f), fill_val, a.dtype),
             jnp.full((80, f), fill_val, a.dtype),
             jnp.full((80, f), fill_val, a.dtype),
             jnp.full((96, f), fill_val, a.dtype)]
    return _pad_rows(parts, fills)


def _flat_idx(idx_list, remap):
    segs = []
    for d, cnt, cnt_pad in zip(DEGS, CNTS, CNTS_PAD):
        idx = idx_list[d - 1].astype(jnp.int32)
        if remap:
            idx = _remap(idx)
        idx = jnp.concatenate(
            [idx, jnp.zeros((cnt_pad - cnt, d), jnp.int32)], axis=0)
        segs.append(idx.reshape(-1))
    return jnp.concatenate(segs, axis=0)


def _prep_wout(w, b, f):
    wt = jnp.zeros((FPAD, FP), jnp.float32).at[:f].set(w.T)
    return wt, b.reshape(1, FP)


def _prep_layer(pd, i, f):
    woutT, bout = _prep_wout(pd["W_out%d" % i], pd["b_out%d" % i], f)
    wselfT = jnp.zeros((FPAD, FPAD), jnp.float32).at[:f, :100].set(
        pd["W_self%d" % i].T)
    bself = jnp.zeros((1, FPAD), jnp.float32).at[0, :100].set(
        pd["b_self%d" % i])
    wx, wb, bd = [], [], []
    for d in DEGS:
        w = pd["W_deg%d_%d" % (i, d)]
        wx.append(jnp.zeros((FPAD, FPAD), jnp.float32).at[:f, :100].set(
            w[:, :f].T))
        wb.append(jnp.zeros((BFPAD, FPAD), jnp.float32).at[:6, :100].set(
            w[:, f:].T))
        bd.append(jnp.zeros((1, FPAD), jnp.float32).at[0, :100].set(
            pd["b_deg%d_%d" % (i, d)]))
    return (woutT, bout, wselfT, bself, jnp.stack(wx), jnp.stack(wb),
            jnp.stack(bd))


def kernel(atom_features, bond_features, atom_neighbors_1, atom_neighbors_2,
           atom_neighbors_3, atom_neighbors_4, bond_neighbors_1,
           bond_neighbors_2, bond_neighbors_3, bond_neighbors_4, mol_ids,
           W_self0, b_self0, W_deg0_1, b_deg0_1, W_deg0_2, b_deg0_2,
           W_deg0_3, b_deg0_3, W_deg0_4, b_deg0_4, W_self1, b_self1,
           W_deg1_1, b_deg1_1, W_deg1_2, b_deg1_2, W_deg1_3, b_deg1_3,
           W_deg1_4, b_deg1_4, W_out0, b_out0, W_out1, b_out1, W_out2,
           b_out2):
    pd = dict(W_self0=W_self0, b_self0=b_self0, W_self1=W_self1,
              b_self1=b_self1, W_out0=W_out0, b_out0=b_out0, W_out1=W_out1,
              b_out1=b_out1, W_out2=W_out2, b_out2=b_out2,
              W_deg0_1=W_deg0_1, b_deg0_1=b_deg0_1, W_deg0_2=W_deg0_2,
              b_deg0_2=b_deg0_2, W_deg0_3=W_deg0_3, b_deg0_3=b_deg0_3,
              W_deg0_4=W_deg0_4, b_deg0_4=b_deg0_4,
              W_deg1_1=W_deg1_1, b_deg1_1=b_deg1_1, W_deg1_2=W_deg1_2,
              b_deg1_2=b_deg1_2, W_deg1_3=W_deg1_3, b_deg1_3=b_deg1_3,
              W_deg1_4=W_deg1_4, b_deg1_4=b_deg1_4)

    x0 = jnp.pad(_pad_atom_rows(atom_features), ((0, 0), (0, FPAD - 62)))
    bf = jnp.pad(bond_features, ((0, 0), (0, BFPAD - 6)))
    aidx = _flat_idx([atom_neighbors_1, atom_neighbors_2, atom_neighbors_3,
                      atom_neighbors_4], remap=True)
    bidx = _flat_idx([bond_neighbors_1, bond_neighbors_2, bond_neighbors_3,
                      bond_neighbors_4], remap=False)
    pad_ids = [4000 + (jnp.arange(n, dtype=jnp.int32) % 16)
               for n in (96, 80, 80, 96)]
    mi = mol_ids.astype(jnp.int32)
    ids = _pad_rows([mi[0:20000], mi[20000:50000], mi[50000:80000],
                     mi[80000:100000]], pad_ids)

    p0 = _prep_layer(pd, 0, 62)
    p1 = _prep_layer(pd, 1, 100)
    woutT2, bout2 = _prep_wout(W_out2, b_out2, 100)

    gb = _bond_gather(bf, bidx)
    gx0 = _atom_gather(x0, aidx)
    o0, x1 = _dense_layer(x0, gx0, gb, *p0)
    gx1 = _atom_gather(x1, aidx)
    o1, x2 = _dense_layer(x1, gx1, gb, *p1)
    o2 = _dense_out(x2, woutT2, bout2)

    fp4 = _fp_scatter(o0, o1, o2, ids)
    return jnp.concatenate([fp4[k, :N_MOLS] for k in range(4)], axis=1)


# speedup vs baseline: 1.6315x; 1.2463x over previous
import functools

import jax
import jax.numpy as jnp
from jax import lax
from jax.experimental import pallas as pl
from jax.experimental.pallas import tpu as pltpu
from jax.experimental.pallas import tpu_sc as plsc

N_ATOMS = 100000
N_BONDS = 200000
N_MOLS = 4000
DEGS = (1, 2, 3, 4)
CNTS = (20000, 30000, 30000, 20000)
CNTS_PAD = (20480, 30720, 30720, 20480)
STARTS = (0, 20480, 51200, 81920)
NP = 102400
TILE = 512
TILES = NP // TILE
TILE_BOUNDS = (40, 100, 160)
REAL_ENDS = (20000, 50480, 81200, 101920)
FP = 512
FPAD = 128
BFPAD = 16
FP_ROWS = 4016
CHUNK = 64
NCHUNKS = tuple(c // CHUNK for c in CNTS_PAD)
IDX_OFFS = (0, 20480, 81920, 174080)
NW = 32
SC_MESH = plsc.VectorSubcoreMesh(core_axis_name="c", subcore_axis_name="s")


def _gather_body(tab, idx_hbm, out_hbm, ibufs, rbufs, obufs, semi, semg,
                 semo, feat):
    wid = lax.axis_index("c") * 16 + lax.axis_index("s")
    for bi, d in enumerate(DEGS):
        nmax = (NCHUNKS[bi] + NW - 1) // NW
        nrows = CHUNK * d

        def valid(k):
            return wid + k * NW < NCHUNKS[bi]

        def cid(k):
            return wid + k * NW

        def idx_copy(k, p):
            return pltpu.make_async_copy(
                idx_hbm.at[pl.ds(IDX_OFFS[bi] + cid(k) * nrows, nrows)],
                ibufs[2 * bi + p], semi)

        def gat_copy(k, p):
            return pltpu.make_async_copy(tab.at[ibufs[2 * bi + p]],
                                         rbufs[p].at[pl.ds(0, nrows)], semg)

        def out_copy(k, p):
            return pltpu.make_async_copy(
                obufs[p], out_hbm.at[pl.ds(STARTS[bi] + cid(k) * CHUNK,
                                           CHUNK)], semo)

        @pl.when(valid(0))
        def _():
            idx_copy(0, 0).start()
            idx_copy(0, 0).wait()
            gat_copy(0, 0).start()

        for k in range(nmax):
            p = k % 2
            q = (k + 1) % 2
            if k + 1 < nmax:
                @pl.when(valid(k + 1))
                def _(k=k, q=q):
                    idx_copy(k + 1, q).start()

            @pl.when(valid(k))
            def _(k=k, p=p):
                gat_copy(k, p).wait()

            if k + 1 < nmax:
                @pl.when(valid(k + 1))
                def _(k=k, q=q):
                    idx_copy(k + 1, q).wait()
                    gat_copy(k + 1, q).start()

            @pl.when(valid(k))
            def _(k=k, p=p, d=d):
                if k >= 2:
                    out_copy(k - 2, p).wait()
                rbuf = rbufs[p]
                obuf = obufs[p]

                def row_body(r, _):
                    for kk in range(feat // 16):
                        acc = rbuf[r * d, pl.ds(kk * 16, 16)]
                        for j in range(1, d):
                            acc = acc + rbuf[r * d + j, pl.ds(kk * 16, 16)]
                        obuf[r, pl.ds(kk * 16, 16)] = acc
                    return 0

                lax.fori_loop(0, CHUNK, row_body, 0)
                out_copy(k, p).start()

        for k in (nmax - 2, nmax - 1):
            if k >= 0:
                @pl.when(valid(k))
                def _(k=k):
                    out_copy(k, k % 2).wait()


def _gather_scratch(feat):
    scratch = []
    for d in DEGS:
        for _ in range(2):
            scratch.append(pltpu.VMEM((CHUNK * d,), jnp.int32))
    for _ in range(2):
        scratch.append(pltpu.VMEM((CHUNK * DEGS[-1], feat), jnp.float32))
    for _ in range(2):
        scratch.append(pltpu.VMEM((CHUNK, feat), jnp.float32))
    scratch.append(pltpu.SemaphoreType.DMA)
    scratch.append(pltpu.SemaphoreType.DMA)
    scratch.append(pltpu.SemaphoreType.DMA)
    return tuple(scratch)


def _split_gather_refs(rest):
    ibufs = rest[:8]
    rbufs = rest[8:10]
    obufs = rest[10:12]
    semi, semg, semo = rest[12:15]
    return ibufs, rbufs, obufs, semi, semg, semo


def _atom_gather(table, aidx):
    def body(tab, idx, out, *rest):
        _gather_body(tab, idx, out, *_split_gather_refs(rest), FPAD)

    fn = pl.kernel(body, out_type=jax.ShapeDtypeStruct((NP, FPAD),
                                                       jnp.float32),
                   mesh=SC_MESH, scratch_types=_gather_scratch(FPAD))
    return fn(table, aidx)


def _bond_gather(btable, bidx):
    def body(tab, idx, out, *rest):
        _gather_body(tab, idx, out, *_split_gather_refs(rest), BFPAD)

    fn = pl.kernel(body, out_type=jax.ShapeDtypeStruct((NP, BFPAD),
                                                       jnp.float32),
                   mesh=SC_MESH, scratch_types=_gather_scratch(BFPAD),
                   compiler_params=pltpu.CompilerParams(
                       use_tc_tiling_on_sc=False))
    return fn(btable, bidx)


SCHUNK = 128


def _fp_scatter(o0, o1, o2, ids):
    chunks_per_s = NP // SCHUNK // 16

    def body(o0r, o1r, o2r, idr, fp_out, ob0, ob1, idbuf, zbuf, fa, fb, sem):
        c = lax.axis_index("c")
        s = lax.axis_index("s")

        z = jnp.zeros((16,), jnp.float32)

        def zrow(r, _):
            for kk in range(128 // 16):
                zbuf[r, pl.ds(kk * 16, 16)] = z
            return 0

        lax.fori_loop(0, 128, zrow, 0)
        base = s * 256
        for fp_sh in (fa, fb):
            @pl.when(s < 15)
            def _(fp_sh=fp_sh):
                pltpu.async_copy(zbuf, fp_sh.at[pl.ds(base, 128)], sem).wait()
                pltpu.async_copy(zbuf, fp_sh.at[pl.ds(base + 128, 128)],
                                 sem).wait()

            @pl.when(s == 15)
            def _(fp_sh=fp_sh):
                pltpu.async_copy(zbuf, fp_sh.at[pl.ds(3840, 128)], sem).wait()
                pltpu.async_copy(zbuf.at[pl.ds(0, 48)],
                                 fp_sh.at[pl.ds(3968, 48)], sem).wait()

        plsc.subcore_barrier()

        srcs = []
        for o_hbm in (o0r, o1r, o2r):
            for local, fp_sh in ((0, fa), (1, fb)):
                srcs.append((o_hbm, local, fp_sh))

        def load(o_hbm, local, row0, buf):
            return pltpu.async_copy(o_hbm.at[2 * c + local].at[
                pl.ds(row0, SCHUNK)], buf, sem)

        def chunk_body(k, _):
            row0 = (s * chunks_per_s + k) * SCHUNK
            pltpu.async_copy(idr.at[pl.ds(row0, SCHUNK)], idbuf, sem).wait()
            bufs = (ob0, ob1)
            load(srcs[0][0], srcs[0][1], row0, bufs[0]).wait()
            for i, (o_hbm, local, fp_sh) in enumerate(srcs):
                nxt = None
                if i + 1 < len(srcs):
                    nxt = load(srcs[i + 1][0], srcs[i + 1][1], row0,
                               bufs[(i + 1) % 2])
                pltpu.sync_copy(bufs[i % 2], fp_sh.at[idbuf], add=True)
                if nxt is not None:
                    nxt.wait()
            return 0

        lax.fori_loop(0, chunks_per_s, chunk_body, 0)
        plsc.subcore_barrier()

        for local, fp_sh in ((0, fa), (1, fb)):
            @pl.when(s < 15)
            def _(local=local, fp_sh=fp_sh):
                pltpu.async_copy(fp_sh.at[pl.ds(base, 256)],
                                 fp_out.at[2 * c + local].at[
                                     pl.ds(base, 256)], sem).wait()

            @pl.when(s == 15)
            def _(local=local, fp_sh=fp_sh):
                pltpu.async_copy(fp_sh.at[pl.ds(3840, 176)],
                                 fp_out.at[2 * c + local].at[
                                     pl.ds(3840, 176)], sem).wait()

    fn = pl.kernel(
        body,
        out_type=jax.ShapeDtypeStruct((4, FP_ROWS, 128), jnp.float32),
        mesh=SC_MESH,
        scratch_types=(
            pltpu.VMEM((SCHUNK, 128), jnp.float32),
            pltpu.VMEM((SCHUNK, 128), jnp.float32),
            pltpu.VMEM((SCHUNK,), jnp.int32),
            pltpu.VMEM((128, 128), jnp.float32),
            pltpu.VMEM_SHARED((FP_ROWS, 128), jnp.float32),
            pltpu.VMEM_SHARED((FP_ROWS, 128), jnp.float32),
            pltpu.SemaphoreType.DMA,
        ),
        compiler_params=pltpu.CompilerParams(use_tc_tiling_on_sc=False),
    )
    return fn(o0, o1, o2, ids)


def _bucket_of(t):
    return ((t >= TILE_BOUNDS[0]).astype(jnp.int32)
            + (t >= TILE_BOUNDS[1]).astype(jnp.int32)
            + (t >= TILE_BOUNDS[2]).astype(jnp.int32))


def _layer_body(x_ref, gx_ref, gb_ref, woutT, bout, wselfT, bself, wxT, wbT,
                bdeg, o_ref, xn_ref):
    t = pl.program_id(0)
    x = x_ref[...]
    logits = jnp.dot(x, woutT[...], preferred_element_type=jnp.float32)
    logits = logits + bout[...]
    m = jnp.max(logits, axis=1, keepdims=True)
    e = jnp.exp(logits - m)
    o = e / jnp.sum(e, axis=1, keepdims=True)
    for k in range(4):
        o_ref[k] = o[:, k * 128:(k + 1) * 128]

    selfl = jnp.dot(x, wselfT[...], preferred_element_type=jnp.float32)
    nb = jnp.dot(gx_ref[...], wxT[0], preferred_element_type=jnp.float32)
    nb = nb + jnp.dot(gb_ref[...], wbT[0], preferred_element_type=jnp.float32)
    tot = nb + bdeg[0] + selfl + bself[...]
    nrm = jnp.sqrt(jnp.sum(tot * tot, axis=1, keepdims=True))
    xn = jnp.maximum(tot / jnp.maximum(nrm, 1e-12), 0.0)

    row = t * TILE + lax.broadcasted_iota(jnp.int32, (TILE, 1), 0)
    b = _bucket_of(t)
    re = jnp.where(b == 0, REAL_ENDS[0],
                   jnp.where(b == 1, REAL_ENDS[1],
                             jnp.where(b == 2, REAL_ENDS[2], REAL_ENDS[3])))
    xn_ref[...] = jnp.where(row < re, xn, 0.0)


def _dense_layer(x, gx, gb, woutT, bout, wselfT, bself, wxT, wbT, bdeg):
    wmap = lambda t: (_bucket_of(t), 0, 0)
    return pl.pallas_call(
        _layer_body,
        grid=(TILES,),
        in_specs=[
            pl.BlockSpec((TILE, FPAD), lambda t: (t, 0)),
            pl.BlockSpec((TILE, FPAD), lambda t: (t, 0)),
            pl.BlockSpec((TILE, BFPAD), lambda t: (t, 0)),
            pl.BlockSpec((FPAD, FP), lambda t: (0, 0)),
            pl.BlockSpec((1, FP), lambda t: (0, 0)),
            pl.BlockSpec((FPAD, FPAD), lambda t: (0, 0)),
            pl.BlockSpec((1, FPAD), lambda t: (0, 0)),
            pl.BlockSpec((1, FPAD, FPAD), wmap),
            pl.BlockSpec((1, BFPAD, FPAD), wmap),
            pl.BlockSpec((1, 1, FPAD), wmap),
        ],
        out_specs=[
            pl.BlockSpec((4, TILE, 128), lambda t: (0, t, 0)),
            pl.BlockSpec((TILE, FPAD), lambda t: (t, 0)),
        ],
        out_shape=[
            jax.ShapeDtypeStruct((4, NP, 128), jnp.float32),
            jax.ShapeDtypeStruct((NP, FPAD), jnp.float32),
        ],
    )(x, gx, gb, woutT, bout, wselfT, bself, wxT, wbT, bdeg)


def _out_body(x_ref, woutT, bout, o_ref):
    logits = jnp.dot(x_ref[...], woutT[...], preferred_element_type=jnp.float32)
    logits = logits + bout[...]
    m = jnp.max(logits, axis=1, keepdims=True)
    e = jnp.exp(logits - m)
    o = e / jnp.sum(e, axis=1, keepdims=True)
    for k in range(4):
        o_ref[k] = o[:, k * 128:(k + 1) * 128]


def _dense_out(x, woutT, bout):
    return pl.pallas_call(
        _out_body,
        grid=(TILES,),
        in_specs=[
            pl.BlockSpec((TILE, FPAD), lambda t: (t, 0)),
            pl.BlockSpec((FPAD, FP), lambda t: (0, 0)),
            pl.BlockSpec((1, FP), lambda t: (0, 0)),
        ],
        out_specs=pl.BlockSpec((4, TILE, 128), lambda t: (0, t, 0)),
        out_shape=jax.ShapeDtypeStruct((4, NP, 128), jnp.float32),
    )(x, woutT, bout)


def _remap(a):
    a = a.astype(jnp.int32)
    return (a + 480 * (a >= 20000).astype(jnp.int32)
            + 720 * (a >= 50000).astype(jnp.int32)
            + 720 * (a >= 80000).astype(jnp.int32))


def _pad_rows(parts, fills):
    segs = []
    for part, fill in zip(parts, fills):
        segs.append(part)
        segs.append(fill)
    return jnp.concatenate(segs, axis=0)


def _pad_atom_rows(a, fill_val=0.0):
    f = a.shape[1]
    parts = [a[0:20000], a[20000:50000], a[50000:80000], a[80000:100000]]
    fills = [jnp.full((480, f), fill_val, a.dtype),
             jnp.full((720, f), fill_val, a.dtype),
             jnp.full((720, f), fill_val, a.dtype),
             jnp.full((480, f), fill_val, a.dtype)]
    return _pad_rows(parts, fills)


def _flat_idx(idx_list, remap):
    segs = []
    for d, cnt, cnt_pad in zip(DEGS, CNTS, CNTS_PAD):
        idx = idx_list[d - 1].astype(jnp.int32)
        if remap:
            idx = _remap(idx)
        idx = jnp.concatenate(
            [idx, jnp.zeros((cnt_pad - cnt, d), jnp.int32)], axis=0)
        segs.append(idx.reshape(-1))
    return jnp.concatenate(segs, axis=0)


def _prep_wout(w, b, f):
    wt = jnp.zeros((FPAD, FP), jnp.float32).at[:f].set(w.T)
    return wt, b.reshape(1, FP)


def _prep_layer(pd, i, f):
    woutT, bout = _prep_wout(pd["W_out%d" % i], pd["b_out%d" % i], f)
    wselfT = jnp.zeros((FPAD, FPAD), jnp.float32).at[:f, :100].set(
        pd["W_self%d" % i].T)
    bself = jnp.zeros((1, FPAD), jnp.float32).at[0, :100].set(
        pd["b_self%d" % i])
    wx, wb, bd = [], [], []
    for d in DEGS:
        w = pd["W_deg%d_%d" % (i, d)]
        wx.append(jnp.zeros((FPAD, FPAD), jnp.float32).at[:f, :100].set(
            w[:, :f].T))
        wb.append(jnp.zeros((BFPAD, FPAD), jnp.float32).at[:6, :100].set(
            w[:, f:].T))
        bd.append(jnp.zeros((1, FPAD), jnp.float32).at[0, :100].set(
            pd["b_deg%d_%d" % (i, d)]))
    return (woutT, bout, wselfT, bself, jnp.stack(wx), jnp.stack(wb),
            jnp.stack(bd))


def kernel(atom_features, bond_features, atom_neighbors_1, atom_neighbors_2,
           atom_neighbors_3, atom_neighbors_4, bond_neighbors_1,
           bond_neighbors_2, bond_neighbors_3, bond_neighbors_4, mol_ids,
           W_self0, b_self0, W_deg0_1, b_deg0_1, W_deg0_2, b_deg0_2,
           W_deg0_3, b_deg0_3, W_deg0_4, b_deg0_4, W_self1, b_self1,
           W_deg1_1, b_deg1_1, W_deg1_2, b_deg1_2, W_deg1_3, b_deg1_3,
           W_deg1_4, b_deg1_4, W_out0, b_out0, W_out1, b_out1, W_out2,
           b_out2):
    pd = dict(W_self0=W_self0, b_self0=b_self0, W_self1=W_self1,
              b_self1=b_self1, W_out0=W_out0, b_out0=b_out0, W_out1=W_out1,
              b_out1=b_out1, W_out2=W_out2, b_out2=b_out2,
              W_deg0_1=W_deg0_1, b_deg0_1=b_deg0_1, W_deg0_2=W_deg0_2,
              b_deg0_2=b_deg0_2, W_deg0_3=W_deg0_3, b_deg0_3=b_deg0_3,
              W_deg0_4=W_deg0_4, b_deg0_4=b_deg0_4,
              W_deg1_1=W_deg1_1, b_deg1_1=b_deg1_1, W_deg1_2=W_deg1_2,
              b_deg1_2=b_deg1_2, W_deg1_3=W_deg1_3, b_deg1_3=b_deg1_3,
              W_deg1_4=W_deg1_4, b_deg1_4=b_deg1_4)

    x0 = jnp.pad(_pad_atom_rows(atom_features), ((0, 0), (0, FPAD - 62)))
    bf = jnp.pad(bond_features, ((0, 0), (0, BFPAD - 6)))
    aidx = _flat_idx([atom_neighbors_1, atom_neighbors_2, atom_neighbors_3,
                      atom_neighbors_4], remap=True)
    bidx = _flat_idx([bond_neighbors_1, bond_neighbors_2, bond_neighbors_3,
                      bond_neighbors_4], remap=False)
    pad_ids = [4000 + (jnp.arange(n, dtype=jnp.int32) % 16)
               for n in (480, 720, 720, 480)]
    mi = mol_ids.astype(jnp.int32)
    ids = _pad_rows([mi[0:20000], mi[20000:50000], mi[50000:80000],
                     mi[80000:100000]], pad_ids)

    p0 = _prep_layer(pd, 0, 62)
    p1 = _prep_layer(pd, 1, 100)
    woutT2, bout2 = _prep_wout(W_out2, b_out2, 100)

    gb = _bond_gather(bf, bidx)
    gx0 = _atom_gather(x0, aidx)
    o0, x1 = _dense_layer(x0, gx0, gb, *p0)
    gx1 = _atom_gather(x1, aidx)
    o1, x2 = _dense_layer(x1, gx1, gb, *p1)
    o2 = _dense_out(x2, woutT2, bout2)

    fp4 = _fp_scatter(o0, o1, o2, ids)
    return jnp.concatenate([fp4[k, :N_MOLS] for k in range(4)], axis=1)


# speedup vs baseline: 1.7670x; 1.0831x over previous
import functools

import jax
import jax.numpy as jnp
from jax import lax
from jax.experimental import pallas as pl
from jax.experimental.pallas import tpu as pltpu
from jax.experimental.pallas import tpu_sc as plsc

N_ATOMS = 100000
N_BONDS = 200000
N_MOLS = 4000
DEGS = (1, 2, 3, 4)
CNTS = (20000, 30000, 30000, 20000)
CNTS_PAD = (20480, 30720, 30720, 20480)
STARTS = (0, 20480, 51200, 81920)
NP = 102400
TILE = 1024
TILES = NP // TILE
TILE_BOUNDS = (20, 50, 80)
REAL_ENDS = (20000, 50480, 81200, 101920)
FP = 512
FPAD = 128
BFPAD = 16
FP_ROWS = 4016
CHUNK = 64
NCHUNKS = tuple(c // CHUNK for c in CNTS_PAD)
IDX_OFFS = (0, 20480, 81920, 174080)
NW = 32
SC_MESH = plsc.VectorSubcoreMesh(core_axis_name="c", subcore_axis_name="s")


def _gather_body(tab, idx_hbm, out_hbm, ibufs, rbufs, obufs, semi, semg,
                 semo, feat):
    wid = lax.axis_index("c") * 16 + lax.axis_index("s")
    for bi, d in enumerate(DEGS):
        nmax = (NCHUNKS[bi] + NW - 1) // NW
        nrows = CHUNK * d

        def valid(k):
            return wid + k * NW < NCHUNKS[bi]

        def cid(k):
            return wid + k * NW

        def idx_copy(k, p):
            return pltpu.make_async_copy(
                idx_hbm.at[pl.ds(IDX_OFFS[bi] + cid(k) * nrows, nrows)],
                ibufs[2 * bi + p], semi)

        def gat_copy(k, p):
            return pltpu.make_async_copy(tab.at[ibufs[2 * bi + p]],
                                         rbufs[p].at[pl.ds(0, nrows)], semg)

        def out_copy(k, p):
            return pltpu.make_async_copy(
                obufs[p], out_hbm.at[pl.ds(STARTS[bi] + cid(k) * CHUNK,
                                           CHUNK)], semo)

        @pl.when(valid(0))
        def _():
            idx_copy(0, 0).start()
            idx_copy(0, 0).wait()
            gat_copy(0, 0).start()

        for k in range(nmax):
            p = k % 2
            q = (k + 1) % 2
            if k + 1 < nmax:
                @pl.when(valid(k + 1))
                def _(k=k, q=q):
                    idx_copy(k + 1, q).start()

            @pl.when(valid(k))
            def _(k=k, p=p):
                gat_copy(k, p).wait()

            if k + 1 < nmax:
                @pl.when(valid(k + 1))
                def _(k=k, q=q):
                    idx_copy(k + 1, q).wait()
                    gat_copy(k + 1, q).start()

            @pl.when(valid(k))
            def _(k=k, p=p, d=d):
                if k >= 2:
                    out_copy(k - 2, p).wait()
                rbuf = rbufs[p]
                obuf = obufs[p]

                def row_body(r, _):
                    for kk in range(feat // 16):
                        acc = rbuf[r * d, pl.ds(kk * 16, 16)]
                        for j in range(1, d):
                            acc = acc + rbuf[r * d + j, pl.ds(kk * 16, 16)]
                        obuf[r, pl.ds(kk * 16, 16)] = acc
                    return 0

                lax.fori_loop(0, CHUNK, row_body, 0)
                out_copy(k, p).start()

        for k in (nmax - 2, nmax - 1):
            if k >= 0:
                @pl.when(valid(k))
                def _(k=k):
                    out_copy(k, k % 2).wait()


def _gather_scratch(feat):
    scratch = []
    for d in DEGS:
        for _ in range(2):
            scratch.append(pltpu.VMEM((CHUNK * d,), jnp.int32))
    for _ in range(2):
        scratch.append(pltpu.VMEM((CHUNK * DEGS[-1], feat), jnp.float32))
    for _ in range(2):
        scratch.append(pltpu.VMEM((CHUNK, feat), jnp.float32))
    scratch.append(pltpu.SemaphoreType.DMA)
    scratch.append(pltpu.SemaphoreType.DMA)
    scratch.append(pltpu.SemaphoreType.DMA)
    return tuple(scratch)


def _split_gather_refs(rest):
    ibufs = rest[:8]
    rbufs = rest[8:10]
    obufs = rest[10:12]
    semi, semg, semo = rest[12:15]
    return ibufs, rbufs, obufs, semi, semg, semo


def _atom_gather(table, aidx):
    def body(tab, idx, out, *rest):
        _gather_body(tab, idx, out, *_split_gather_refs(rest), FPAD)

    fn = pl.kernel(body, out_type=jax.ShapeDtypeStruct((NP, FPAD),
                                                       jnp.float32),
                   mesh=SC_MESH, scratch_types=_gather_scratch(FPAD))
    return fn(table, aidx)


def _bond_gather(btable, bidx):
    def body(tab, idx, out, *rest):
        _gather_body(tab, idx, out, *_split_gather_refs(rest), BFPAD)

    fn = pl.kernel(body, out_type=jax.ShapeDtypeStruct((NP, BFPAD),
                                                       jnp.float32),
                   mesh=SC_MESH, scratch_types=_gather_scratch(BFPAD),
                   compiler_params=pltpu.CompilerParams(
                       use_tc_tiling_on_sc=False))
    return fn(btable, bidx)


SCHUNK = 128


def _fp_scatter(o0, o1, o2, ids):
    chunks_per_s = NP // SCHUNK // 16

    def body(o0r, o1r, o2r, idr, fp_out, ob0, ob1, idbuf, zbuf, fa, fb, sem):
        c = lax.axis_index("c")
        s = lax.axis_index("s")

        z = jnp.zeros((16,), jnp.float32)

        def zrow(r, _):
            for kk in range(128 // 16):
                zbuf[r, pl.ds(kk * 16, 16)] = z
            return 0

        lax.fori_loop(0, 128, zrow, 0)
        base = s * 256
        for fp_sh in (fa, fb):
            @pl.when(s < 15)
            def _(fp_sh=fp_sh):
                pltpu.async_copy(zbuf, fp_sh.at[pl.ds(base, 128)], sem).wait()
                pltpu.async_copy(zbuf, fp_sh.at[pl.ds(base + 128, 128)],
                                 sem).wait()

            @pl.when(s == 15)
            def _(fp_sh=fp_sh):
                pltpu.async_copy(zbuf, fp_sh.at[pl.ds(3840, 128)], sem).wait()
                pltpu.async_copy(zbuf.at[pl.ds(0, 48)],
                                 fp_sh.at[pl.ds(3968, 48)], sem).wait()

        plsc.subcore_barrier()

        srcs = []
        for o_hbm in (o0r, o1r, o2r):
            for local, fp_sh in ((0, fa), (1, fb)):
                srcs.append((o_hbm, local, fp_sh))

        def load(o_hbm, local, row0, buf):
            return pltpu.async_copy(o_hbm.at[2 * c + local].at[
                pl.ds(row0, SCHUNK)], buf, sem)

        def chunk_body(k, _):
            row0 = (s * chunks_per_s + k) * SCHUNK
            pltpu.async_copy(idr.at[pl.ds(row0, SCHUNK)], idbuf, sem).wait()
            bufs = (ob0, ob1)
            load(srcs[0][0], srcs[0][1], row0, bufs[0]).wait()
            for i, (o_hbm, local, fp_sh) in enumerate(srcs):
                nxt = None
                if i + 1 < len(srcs):
                    nxt = load(srcs[i + 1][0], srcs[i + 1][1], row0,
                               bufs[(i + 1) % 2])
                pltpu.sync_copy(bufs[i % 2], fp_sh.at[idbuf], add=True)
                if nxt is not None:
                    nxt.wait()
            return 0

        lax.fori_loop(0, chunks_per_s, chunk_body, 0)
        plsc.subcore_barrier()

        for local, fp_sh in ((0, fa), (1, fb)):
            @pl.when(s < 15)
            def _(local=local, fp_sh=fp_sh):
                pltpu.async_copy(fp_sh.at[pl.ds(base, 256)],
                                 fp_out.at[2 * c + local].at[
                                     pl.ds(base, 256)], sem).wait()

            @pl.when(s == 15)
            def _(local=local, fp_sh=fp_sh):
                pltpu.async_copy(fp_sh.at[pl.ds(3840, 176)],
                                 fp_out.at[2 * c + local].at[
                                     pl.ds(3840, 176)], sem).wait()

    fn = pl.kernel(
        body,
        out_type=jax.ShapeDtypeStruct((4, FP_ROWS, 128), jnp.float32),
        mesh=SC_MESH,
        scratch_types=(
            pltpu.VMEM((SCHUNK, 128), jnp.float32),
            pltpu.VMEM((SCHUNK, 128), jnp.float32),
            pltpu.VMEM((SCHUNK,), jnp.int32),
            pltpu.VMEM((128, 128), jnp.float32),
            pltpu.VMEM_SHARED((FP_ROWS, 128), jnp.float32),
            pltpu.VMEM_SHARED((FP_ROWS, 128), jnp.float32),
            pltpu.SemaphoreType.DMA,
        ),
        compiler_params=pltpu.CompilerParams(use_tc_tiling_on_sc=False),
    )
    return fn(o0, o1, o2, ids)


def _bucket_of(t):
    return ((t >= TILE_BOUNDS[0]).astype(jnp.int32)
            + (t >= TILE_BOUNDS[1]).astype(jnp.int32)
            + (t >= TILE_BOUNDS[2]).astype(jnp.int32))


def _layer_body(x_ref, gx_ref, gb_ref, woutT, bout, wselfT, bself, wxT, wbT,
                bdeg, o_ref, xn_ref):
    t = pl.program_id(0)
    x = x_ref[...]
    logits = jnp.dot(x, woutT[...], preferred_element_type=jnp.float32)
    logits = logits + bout[...]
    m = jnp.max(logits, axis=1, keepdims=True)
    e = jnp.exp(logits - m)
    o = e / jnp.sum(e, axis=1, keepdims=True)
    for k in range(4):
        o_ref[k] = o[:, k * 128:(k + 1) * 128]

    selfl = jnp.dot(x, wselfT[...], preferred_element_type=jnp.float32)
    nb = jnp.dot(gx_ref[...], wxT[0], preferred_element_type=jnp.float32)
    nb = nb + jnp.dot(gb_ref[...], wbT[0], preferred_element_type=jnp.float32)
    tot = nb + bdeg[0] + selfl + bself[...]
    nrm = jnp.sqrt(jnp.sum(tot * tot, axis=1, keepdims=True))
    xn = jnp.maximum(tot / jnp.maximum(nrm, 1e-12), 0.0)

    row = t * TILE + lax.broadcasted_iota(jnp.int32, (TILE, 1), 0)
    b = _bucket_of(t)
    re = jnp.where(b == 0, REAL_ENDS[0],
                   jnp.where(b == 1, REAL_ENDS[1],
                             jnp.where(b == 2, REAL_ENDS[2], REAL_ENDS[3])))
    xn_ref[...] = jnp.where(row < re, xn, 0.0)


def _dense_layer(x, gx, gb, woutT, bout, wselfT, bself, wxT, wbT, bdeg):
    wmap = lambda t: (_bucket_of(t), 0, 0)
    return pl.pallas_call(
        _layer_body,
        grid=(TILES,),
        in_specs=[
            pl.BlockSpec((TILE, FPAD), lambda t: (t, 0)),
            pl.BlockSpec((TILE, FPAD), lambda t: (t, 0)),
            pl.BlockSpec((TILE, BFPAD), lambda t: (t, 0)),
            pl.BlockSpec((FPAD, FP), lambda t: (0, 0)),
            pl.BlockSpec((1, FP), lambda t: (0, 0)),
            pl.BlockSpec((FPAD, FPAD), lambda t: (0, 0)),
            pl.BlockSpec((1, FPAD), lambda t: (0, 0)),
            pl.BlockSpec((1, FPAD, FPAD), wmap),
            pl.BlockSpec((1, BFPAD, FPAD), wmap),
            pl.BlockSpec((1, 1, FPAD), wmap),
        ],
        out_specs=[
            pl.BlockSpec((4, TILE, 128), lambda t: (0, t, 0)),
            pl.BlockSpec((TILE, FPAD), lambda t: (t, 0)),
        ],
        out_shape=[
            jax.ShapeDtypeStruct((4, NP, 128), jnp.float32),
            jax.ShapeDtypeStruct((NP, FPAD), jnp.float32),
        ],
    )(x, gx, gb, woutT, bout, wselfT, bself, wxT, wbT, bdeg)


def _out_body(x_ref, woutT, bout, o_ref):
    logits = jnp.dot(x_ref[...], woutT[...], preferred_element_type=jnp.float32)
    logits = logits + bout[...]
    m = jnp.max(logits, axis=1, keepdims=True)
    e = jnp.exp(logits - m)
    o = e / jnp.sum(e, axis=1, keepdims=True)
    for k in range(4):
        o_ref[k] = o[:, k * 128:(k + 1) * 128]


def _dense_out(x, woutT, bout):
    return pl.pallas_call(
        _out_body,
        grid=(TILES,),
        in_specs=[
            pl.BlockSpec((TILE, FPAD), lambda t: (t, 0)),
            pl.BlockSpec((FPAD, FP), lambda t: (0, 0)),
            pl.BlockSpec((1, FP), lambda t: (0, 0)),
        ],
        out_specs=pl.BlockSpec((4, TILE, 128), lambda t: (0, t, 0)),
        out_shape=jax.ShapeDtypeStruct((4, NP, 128), jnp.float32),
    )(x, woutT, bout)


def _remap(a):
    a = a.astype(jnp.int32)
    return (a + 480 * (a >= 20000).astype(jnp.int32)
            + 720 * (a >= 50000).astype(jnp.int32)
            + 720 * (a >= 80000).astype(jnp.int32))


def _pad_rows(parts, fills):
    segs = []
    for part, fill in zip(parts, fills):
        segs.append(part)
        segs.append(fill)
    return jnp.concatenate(segs, axis=0)


def _pad_atom_rows(a, fill_val=0.0):
    f = a.shape[1]
    parts = [a[0:20000], a[20000:50000], a[50000:80000], a[80000:100000]]
    fills = [jnp.full((480, f), fill_val, a.dtype),
             jnp.full((720, f), fill_val, a.dtype),
             jnp.full((720, f), fill_val, a.dtype),
             jnp.full((480, f), fill_val, a.dtype)]
    return _pad_rows(parts, fills)


def _flat_idx(idx_list, remap):
    segs = []
    for d, cnt, cnt_pad in zip(DEGS, CNTS, CNTS_PAD):
        idx = idx_list[d - 1].astype(jnp.int32)
        if remap:
            idx = _remap(idx)
        idx = jnp.concatenate(
            [idx, jnp.zeros((cnt_pad - cnt, d), jnp.int32)], axis=0)
        segs.append(idx.reshape(-1))
    return jnp.concatenate(segs, axis=0)


def _prep_wout(w, b, f):
    wt = jnp.zeros((FPAD, FP), jnp.float32).at[:f].set(w.T)
    return wt, b.reshape(1, FP)


def _prep_layer(pd, i, f):
    woutT, bout = _prep_wout(pd["W_out%d" % i], pd["b_out%d" % i], f)
    wselfT = jnp.zeros((FPAD, FPAD), jnp.float32).at[:f, :100].set(
        pd["W_self%d" % i].T)
    bself = jnp.zeros((1, FPAD), jnp.float32).at[0, :100].set(
        pd["b_self%d" % i])
    wx, wb, bd = [], [], []
    for d in DEGS:
        w = pd["W_deg%d_%d" % (i, d)]
        wx.append(jnp.zeros((FPAD, FPAD), jnp.float32).at[:f, :100].set(
            w[:, :f].T))
        wb.append(jnp.zeros((BFPAD, FPAD), jnp.float32).at[:6, :100].set(
            w[:, f:].T))
        bd.append(jnp.zeros((1, FPAD), jnp.float32).at[0, :100].set(
            pd["b_deg%d_%d" % (i, d)]))
    return (woutT, bout, wselfT, bself, jnp.stack(wx), jnp.stack(wb),
            jnp.stack(bd))


def kernel(atom_features, bond_features, atom_neighbors_1, atom_neighbors_2,
           atom_neighbors_3, atom_neighbors_4, bond_neighbors_1,
           bond_neighbors_2, bond_neighbors_3, bond_neighbors_4, mol_ids,
           W_self0, b_self0, W_deg0_1, b_deg0_1, W_deg0_2, b_deg0_2,
           W_deg0_3, b_deg0_3, W_deg0_4, b_deg0_4, W_self1, b_self1,
           W_deg1_1, b_deg1_1, W_deg1_2, b_deg1_2, W_deg1_3, b_deg1_3,
           W_deg1_4, b_deg1_4, W_out0, b_out0, W_out1, b_out1, W_out2,
           b_out2):
    pd = dict(W_self0=W_self0, b_self0=b_self0, W_self1=W_self1,
              b_self1=b_self1, W_out0=W_out0, b_out0=b_out0, W_out1=W_out1,
              b_out1=b_out1, W_out2=W_out2, b_out2=b_out2,
              W_deg0_1=W_deg0_1, b_deg0_1=b_deg0_1, W_deg0_2=W_deg0_2,
              b_deg0_2=b_deg0_2, W_deg0_3=W_deg0_3, b_deg0_3=b_deg0_3,
              W_deg0_4=W_deg0_4, b_deg0_4=b_deg0_4,
              W_deg1_1=W_deg1_1, b_deg1_1=b_deg1_1, W_deg1_2=W_deg1_2,
              b_deg1_2=b_deg1_2, W_deg1_3=W_deg1_3, b_deg1_3=b_deg1_3,
              W_deg1_4=W_deg1_4, b_deg1_4=b_deg1_4)

    x0 = jnp.pad(_pad_atom_rows(atom_features), ((0, 0), (0, FPAD - 62)))
    bf = jnp.pad(bond_features, ((0, 0), (0, BFPAD - 6)))
    aidx = _flat_idx([atom_neighbors_1, atom_neighbors_2, atom_neighbors_3,
                      atom_neighbors_4], remap=True)
    bidx = _flat_idx([bond_neighbors_1, bond_neighbors_2, bond_neighbors_3,
                      bond_neighbors_4], remap=False)
    pad_ids = [4000 + (jnp.arange(n, dtype=jnp.int32) % 16)
               for n in (480, 720, 720, 480)]
    mi = mol_ids.astype(jnp.int32)
    ids = _pad_rows([mi[0:20000], mi[20000:50000], mi[50000:80000],
                     mi[80000:100000]], pad_ids)

    p0 = _prep_layer(pd, 0, 62)
    p1 = _prep_layer(pd, 1, 100)
    woutT2, bout2 = _prep_wout(W_out2, b_out2, 100)

    gb = _bond_gather(bf, bidx)
    gx0 = _atom_gather(x0, aidx)
    o0, x1 = _dense_layer(x0, gx0, gb, *p0)
    gx1 = _atom_gather(x1, aidx)
    o1, x2 = _dense_layer(x1, gx1, gb, *p1)
    o2 = _dense_out(x2, woutT2, bout2)

    fp4 = _fp_scatter(o0, o1, o2, ids)
    return jnp.concatenate([fp4[k, :N_MOLS] for k in range(4)], axis=1)


# speedup vs baseline: 1.8480x; 1.0458x over previous
import functools

import jax
import jax.numpy as jnp
from jax import lax
from jax.experimental import pallas as pl
from jax.experimental.pallas import tpu as pltpu
from jax.experimental.pallas import tpu_sc as plsc

N_ATOMS = 100000
N_BONDS = 200000
N_MOLS = 4000
DEGS = (1, 2, 3, 4)
CNTS = (20000, 30000, 30000, 20000)
CNTS_PAD = (20480, 30720, 30720, 20480)
STARTS = (0, 20480, 51200, 81920)
NP = 102400
TILE = 2048
TILES = NP // TILE
TILE_BOUNDS = (10, 25, 40)
REAL_ENDS = (20000, 50480, 81200, 101920)
FP = 512
FPAD = 128
BFPAD = 16
FP_ROWS = 4016
CHUNK = 64
NCHUNKS = tuple(c // CHUNK for c in CNTS_PAD)
IDX_OFFS = (0, 20480, 81920, 174080)
NW = 32
SC_MESH = plsc.VectorSubcoreMesh(core_axis_name="c", subcore_axis_name="s")


def _gather_body(tab, idx_hbm, out_hbm, ibufs, rbufs, obufs, semi, semg,
                 semo, feat):
    wid = lax.axis_index("c") * 16 + lax.axis_index("s")
    for bi, d in enumerate(DEGS):
        nmax = (NCHUNKS[bi] + NW - 1) // NW
        nrows = CHUNK * d

        def valid(k):
            return wid + k * NW < NCHUNKS[bi]

        def cid(k):
            return wid + k * NW

        def idx_copy(k, p):
            return pltpu.make_async_copy(
                idx_hbm.at[pl.ds(IDX_OFFS[bi] + cid(k) * nrows, nrows)],
                ibufs[2 * bi + p], semi)

        def gat_copy(k, p):
            return pltpu.make_async_copy(tab.at[ibufs[2 * bi + p]],
                                         rbufs[p].at[pl.ds(0, nrows)], semg)

        def out_copy(k, p):
            return pltpu.make_async_copy(
                obufs[p], out_hbm.at[pl.ds(STARTS[bi] + cid(k) * CHUNK,
                                           CHUNK)], semo)

        @pl.when(valid(0))
        def _():
            idx_copy(0, 0).start()
            idx_copy(0, 0).wait()
            gat_copy(0, 0).start()

        for k in range(nmax):
            p = k % 2
            q = (k + 1) % 2
            if k + 1 < nmax:
                @pl.when(valid(k + 1))
                def _(k=k, q=q):
                    idx_copy(k + 1, q).start()

            @pl.when(valid(k))
            def _(k=k, p=p):
                gat_copy(k, p).wait()

            if k + 1 < nmax:
                @pl.when(valid(k + 1))
                def _(k=k, q=q):
                    idx_copy(k + 1, q).wait()
                    gat_copy(k + 1, q).start()

            @pl.when(valid(k))
            def _(k=k, p=p, d=d):
                if k >= 2:
                    out_copy(k - 2, p).wait()
                rbuf = rbufs[p]
                obuf = obufs[p]

                def row_body(r, _):
                    for kk in range(feat // 16):
                        acc = rbuf[r * d, pl.ds(kk * 16, 16)]
                        for j in range(1, d):
                            acc = acc + rbuf[r * d + j, pl.ds(kk * 16, 16)]
                        obuf[r, pl.ds(kk * 16, 16)] = acc
                    return 0

                lax.fori_loop(0, CHUNK, row_body, 0)
                out_copy(k, p).start()

        for k in (nmax - 2, nmax - 1):
            if k >= 0:
                @pl.when(valid(k))
                def _(k=k):
                    out_copy(k, k % 2).wait()


def _gather_scratch(feat):
    scratch = []
    for d in DEGS:
        for _ in range(2):
            scratch.append(pltpu.VMEM((CHUNK * d,), jnp.int32))
    for _ in range(2):
        scratch.append(pltpu.VMEM((CHUNK * DEGS[-1], feat), jnp.float32))
    for _ in range(2):
        scratch.append(pltpu.VMEM((CHUNK, feat), jnp.float32))
    scratch.append(pltpu.SemaphoreType.DMA)
    scratch.append(pltpu.SemaphoreType.DMA)
    scratch.append(pltpu.SemaphoreType.DMA)
    return tuple(scratch)


def _split_gather_refs(rest):
    ibufs = rest[:8]
    rbufs = rest[8:10]
    obufs = rest[10:12]
    semi, semg, semo = rest[12:15]
    return ibufs, rbufs, obufs, semi, semg, semo


def _atom_gather(table, aidx):
    def body(tab, idx, out, *rest):
        _gather_body(tab, idx, out, *_split_gather_refs(rest), FPAD)

    fn = pl.kernel(body, out_type=jax.ShapeDtypeStruct((NP, FPAD),
                                                       jnp.float32),
                   mesh=SC_MESH, scratch_types=_gather_scratch(FPAD))
    return fn(table, aidx)


def _bond_gather(btable, bidx):
    def body(tab, idx, out, *rest):
        _gather_body(tab, idx, out, *_split_gather_refs(rest), BFPAD)

    fn = pl.kernel(body, out_type=jax.ShapeDtypeStruct((NP, BFPAD),
                                                       jnp.float32),
                   mesh=SC_MESH, scratch_types=_gather_scratch(BFPAD),
                   compiler_params=pltpu.CompilerParams(
                       use_tc_tiling_on_sc=False))
    return fn(btable, bidx)


SCHUNK = 128


def _fp_scatter(o0, o1, o2, ids):
    chunks_per_s = NP // SCHUNK // 16

    def body(o0r, o1r, o2r, idr, fp_out, ob0, ob1, idbuf, zbuf, fa, fb, sem):
        c = lax.axis_index("c")
        s = lax.axis_index("s")

        z = jnp.zeros((16,), jnp.float32)

        def zrow(r, _):
            for kk in range(128 // 16):
                zbuf[r, pl.ds(kk * 16, 16)] = z
            return 0

        lax.fori_loop(0, 128, zrow, 0)
        base = s * 256
        for fp_sh in (fa, fb):
            @pl.when(s < 15)
            def _(fp_sh=fp_sh):
                pltpu.async_copy(zbuf, fp_sh.at[pl.ds(base, 128)], sem).wait()
                pltpu.async_copy(zbuf, fp_sh.at[pl.ds(base + 128, 128)],
                                 sem).wait()

            @pl.when(s == 15)
            def _(fp_sh=fp_sh):
                pltpu.async_copy(zbuf, fp_sh.at[pl.ds(3840, 128)], sem).wait()
                pltpu.async_copy(zbuf.at[pl.ds(0, 48)],
                                 fp_sh.at[pl.ds(3968, 48)], sem).wait()

        plsc.subcore_barrier()

        srcs = []
        for o_hbm in (o0r, o1r, o2r):
            for local, fp_sh in ((0, fa), (1, fb)):
                srcs.append((o_hbm, local, fp_sh))

        def load(o_hbm, local, row0, buf):
            return pltpu.async_copy(o_hbm.at[2 * c + local].at[
                pl.ds(row0, SCHUNK)], buf, sem)

        def chunk_body(k, _):
            row0 = (s * chunks_per_s + k) * SCHUNK
            pltpu.async_copy(idr.at[pl.ds(row0, SCHUNK)], idbuf, sem).wait()
            bufs = (ob0, ob1)
            load(srcs[0][0], srcs[0][1], row0, bufs[0]).wait()
            for i, (o_hbm, local, fp_sh) in enumerate(srcs):
                nxt = None
                if i + 1 < len(srcs):
                    nxt = load(srcs[i + 1][0], srcs[i + 1][1], row0,
                               bufs[(i + 1) % 2])
                pltpu.sync_copy(bufs[i % 2], fp_sh.at[idbuf], add=True)
                if nxt is not None:
                    nxt.wait()
            return 0

        lax.fori_loop(0, chunks_per_s, chunk_body, 0)
        plsc.subcore_barrier()

        for local, fp_sh in ((0, fa), (1, fb)):
            @pl.when(s < 15)
            def _(local=local, fp_sh=fp_sh):
                pltpu.async_copy(fp_sh.at[pl.ds(base, 256)],
                                 fp_out.at[2 * c + local].at[
                                     pl.ds(base, 256)], sem).wait()

            @pl.when(s == 15)
            def _(local=local, fp_sh=fp_sh):
                pltpu.async_copy(fp_sh.at[pl.ds(3840, 176)],
                                 fp_out.at[2 * c + local].at[
                                     pl.ds(3840, 176)], sem).wait()

    fn = pl.kernel(
        body,
        out_type=jax.ShapeDtypeStruct((4, FP_ROWS, 128), jnp.float32),
        mesh=SC_MESH,
        scratch_types=(
            pltpu.VMEM((SCHUNK, 128), jnp.float32),
            pltpu.VMEM((SCHUNK, 128), jnp.float32),
            pltpu.VMEM((SCHUNK,), jnp.int32),
            pltpu.VMEM((128, 128), jnp.float32),
            pltpu.VMEM_SHARED((FP_ROWS, 128), jnp.float32),
            pltpu.VMEM_SHARED((FP_ROWS, 128), jnp.float32),
            pltpu.SemaphoreType.DMA,
        ),
        compiler_params=pltpu.CompilerParams(use_tc_tiling_on_sc=False),
    )
    return fn(o0, o1, o2, ids)


def _bucket_of(t):
    return ((t >= TILE_BOUNDS[0]).astype(jnp.int32)
            + (t >= TILE_BOUNDS[1]).astype(jnp.int32)
            + (t >= TILE_BOUNDS[2]).astype(jnp.int32))


def _layer_body(x_ref, gx_ref, gb_ref, woutT, bout, wselfT, bself, wxT, wbT,
                bdeg, o_ref, xn_ref):
    t = pl.program_id(0)
    x = x_ref[...]
    logits = jnp.dot(x, woutT[...], preferred_element_type=jnp.float32)
    logits = logits + bout[...]
    m = jnp.max(logits, axis=1, keepdims=True)
    e = jnp.exp(logits - m)
    o = e / jnp.sum(e, axis=1, keepdims=True)
    for k in range(4):
        o_ref[k] = o[:, k * 128:(k + 1) * 128]

    selfl = jnp.dot(x, wselfT[...], preferred_element_type=jnp.float32)
    nb = jnp.dot(gx_ref[...], wxT[0], preferred_element_type=jnp.float32)
    nb = nb + jnp.dot(gb_ref[...], wbT[0], preferred_element_type=jnp.float32)
    tot = nb + bdeg[0] + selfl + bself[...]
    nrm = jnp.sqrt(jnp.sum(tot * tot, axis=1, keepdims=True))
    xn = jnp.maximum(tot / jnp.maximum(nrm, 1e-12), 0.0)

    row = t * TILE + lax.broadcasted_iota(jnp.int32, (TILE, 1), 0)
    b = _bucket_of(t)
    re = jnp.where(b == 0, REAL_ENDS[0],
                   jnp.where(b == 1, REAL_ENDS[1],
                             jnp.where(b == 2, REAL_ENDS[2], REAL_ENDS[3])))
    xn_ref[...] = jnp.where(row < re, xn, 0.0)


def _dense_layer(x, gx, gb, woutT, bout, wselfT, bself, wxT, wbT, bdeg):
    wmap = lambda t: (_bucket_of(t), 0, 0)
    return pl.pallas_call(
        _layer_body,
        grid=(TILES,),
        in_specs=[
            pl.BlockSpec((TILE, FPAD), lambda t: (t, 0)),
            pl.BlockSpec((TILE, FPAD), lambda t: (t, 0)),
            pl.BlockSpec((TILE, BFPAD), lambda t: (t, 0)),
            pl.BlockSpec((FPAD, FP), lambda t: (0, 0)),
            pl.BlockSpec((1, FP), lambda t: (0, 0)),
            pl.BlockSpec((FPAD, FPAD), lambda t: (0, 0)),
            pl.BlockSpec((1, FPAD), lambda t: (0, 0)),
            pl.BlockSpec((1, FPAD, FPAD), wmap),
            pl.BlockSpec((1, BFPAD, FPAD), wmap),
            pl.BlockSpec((1, 1, FPAD), wmap),
        ],
        out_specs=[
            pl.BlockSpec((4, TILE, 128), lambda t: (0, t, 0)),
            pl.BlockSpec((TILE, FPAD), lambda t: (t, 0)),
        ],
        out_shape=[
            jax.ShapeDtypeStruct((4, NP, 128), jnp.float32),
            jax.ShapeDtypeStruct((NP, FPAD), jnp.float32),
        ],
    )(x, gx, gb, woutT, bout, wselfT, bself, wxT, wbT, bdeg)


def _out_body(x_ref, woutT, bout, o_ref):
    logits = jnp.dot(x_ref[...], woutT[...], preferred_element_type=jnp.float32)
    logits = logits + bout[...]
    m = jnp.max(logits, axis=1, keepdims=True)
    e = jnp.exp(logits - m)
    o = e / jnp.sum(e, axis=1, keepdims=True)
    for k in range(4):
        o_ref[k] = o[:, k * 128:(k + 1) * 128]


def _dense_out(x, woutT, bout):
    return pl.pallas_call(
        _out_body,
        grid=(TILES,),
        in_specs=[
            pl.BlockSpec((TILE, FPAD), lambda t: (t, 0)),
            pl.BlockSpec((FPAD, FP), lambda t: (0, 0)),
            pl.BlockSpec((1, FP), lambda t: (0, 0)),
        ],
        out_specs=pl.BlockSpec((4, TILE, 128), lambda t: (0, t, 0)),
        out_shape=jax.ShapeDtypeStruct((4, NP, 128), jnp.float32),
    )(x, woutT, bout)


def _remap(a):
    a = a.astype(jnp.int32)
    return (a + 480 * (a >= 20000).astype(jnp.int32)
            + 720 * (a >= 50000).astype(jnp.int32)
            + 720 * (a >= 80000).astype(jnp.int32))


def _pad_rows(parts, fills):
    segs = []
    for part, fill in zip(parts, fills):
        segs.append(part)
        segs.append(fill)
    return jnp.concatenate(segs, axis=0)


def _pad_atom_rows(a, fill_val=0.0):
    f = a.shape[1]
    parts = [a[0:20000], a[20000:50000], a[50000:80000], a[80000:100000]]
    fills = [jnp.full((480, f), fill_val, a.dtype),
             jnp.full((720, f), fill_val, a.dtype),
             jnp.full((720, f), fill_val, a.dtype),
             jnp.full((480, f), fill_val, a.dtype)]
    return _pad_rows(parts, fills)


def _flat_idx(idx_list, remap):
    segs = []
    for d, cnt, cnt_pad in zip(DEGS, CNTS, CNTS_PAD):
        idx = idx_list[d - 1].astype(jnp.int32)
        if remap:
            idx = _remap(idx)
        idx = jnp.concatenate(
            [idx, jnp.zeros((cnt_pad - cnt, d), jnp.int32)], axis=0)
        segs.append(idx.reshape(-1))
    return jnp.concatenate(segs, axis=0)


def _prep_wout(w, b, f):
    wt = jnp.zeros((FPAD, FP), jnp.float32).at[:f].set(w.T)
    return wt, b.reshape(1, FP)


def _prep_layer(pd, i, f):
    woutT, bout = _prep_wout(pd["W_out%d" % i], pd["b_out%d" % i], f)
    wselfT = jnp.zeros((FPAD, FPAD), jnp.float32).at[:f, :100].set(
        pd["W_self%d" % i].T)
    bself = jnp.zeros((1, FPAD), jnp.float32).at[0, :100].set(
        pd["b_self%d" % i])
    wx, wb, bd = [], [], []
    for d in DEGS:
        w = pd["W_deg%d_%d" % (i, d)]
        wx.append(jnp.zeros((FPAD, FPAD), jnp.float32).at[:f, :100].set(
            w[:, :f].T))
        wb.append(jnp.zeros((BFPAD, FPAD), jnp.float32).at[:6, :100].set(
            w[:, f:].T))
        bd.append(jnp.zeros((1, FPAD), jnp.float32).at[0, :100].set(
            pd["b_deg%d_%d" % (i, d)]))
    return (woutT, bout, wselfT, bself, jnp.stack(wx), jnp.stack(wb),
            jnp.stack(bd))


def kernel(atom_features, bond_features, atom_neighbors_1, atom_neighbors_2,
           atom_neighbors_3, atom_neighbors_4, bond_neighbors_1,
           bond_neighbors_2, bond_neighbors_3, bond_neighbors_4, mol_ids,
           W_self0, b_self0, W_deg0_1, b_deg0_1, W_deg0_2, b_deg0_2,
           W_deg0_3, b_deg0_3, W_deg0_4, b_deg0_4, W_self1, b_self1,
           W_deg1_1, b_deg1_1, W_deg1_2, b_deg1_2, W_deg1_3, b_deg1_3,
           W_deg1_4, b_deg1_4, W_out0, b_out0, W_out1, b_out1, W_out2,
           b_out2):
    pd = dict(W_self0=W_self0, b_self0=b_self0, W_self1=W_self1,
              b_self1=b_self1, W_out0=W_out0, b_out0=b_out0, W_out1=W_out1,
              b_out1=b_out1, W_out2=W_out2, b_out2=b_out2,
              W_deg0_1=W_deg0_1, b_deg0_1=b_deg0_1, W_deg0_2=W_deg0_2,
              b_deg0_2=b_deg0_2, W_deg0_3=W_deg0_3, b_deg0_3=b_deg0_3,
              W_deg0_4=W_deg0_4, b_deg0_4=b_deg0_4,
              W_deg1_1=W_deg1_1, b_deg1_1=b_deg1_1, W_deg1_2=W_deg1_2,
              b_deg1_2=b_deg1_2, W_deg1_3=W_deg1_3, b_deg1_3=b_deg1_3,
              W_deg1_4=W_deg1_4, b_deg1_4=b_deg1_4)

    x0 = jnp.pad(_pad_atom_rows(atom_features), ((0, 0), (0, FPAD - 62)))
    bf = jnp.pad(bond_features, ((0, 0), (0, BFPAD - 6)))
    aidx = _flat_idx([atom_neighbors_1, atom_neighbors_2, atom_neighbors_3,
                      atom_neighbors_4], remap=True)
    bidx = _flat_idx([bond_neighbors_1, bond_neighbors_2, bond_neighbors_3,
                      bond_neighbors_4], remap=False)
    pad_ids = [4000 + (jnp.arange(n, dtype=jnp.int32) % 16)
               for n in (480, 720, 720, 480)]
    mi = mol_ids.astype(jnp.int32)
    ids = _pad_rows([mi[0:20000], mi[20000:50000], mi[50000:80000],
                     mi[80000:100000]], pad_ids)

    p0 = _prep_layer(pd, 0, 62)
    p1 = _prep_layer(pd, 1, 100)
    woutT2, bout2 = _prep_wout(W_out2, b_out2, 100)

    gb = _bond_gather(bf, bidx)
    gx0 = _atom_gather(x0, aidx)
    o0, x1 = _dense_layer(x0, gx0, gb, *p0)
    gx1 = _atom_gather(x1, aidx)
    o1, x2 = _dense_layer(x1, gx1, gb, *p1)
    o2 = _dense_out(x2, woutT2, bout2)

    fp4 = _fp_scatter(o0, o1, o2, ids)
    return jnp.concatenate([fp4[k, :N_MOLS] for k in range(4)], axis=1)
